# Initial kernel scaffold; baseline (speedup 1.0000x reference)
import functools

import jax
import jax.numpy as jnp
from jax import lax
from jax.experimental import pallas as pl
from jax.experimental.pallas import tpu as pltpu
from jax.experimental.pallas import tpu_sc as plsc

NC = 2
NS = 16
NW = NC * NS
LANES = 16
B_EDGE = 128
NBUF = 2


def _round_up(x, m):
    return (x + m - 1) // m * m


def _make_deg_kernel(NP, K):
    R = NP // NS
    mesh = plsc.VectorSubcoreMesh(core_axis_name="c", subcore_axis_name="s")

    @functools.partial(
        pl.kernel,
        out_type=jax.ShapeDtypeStruct((NC, NP), jnp.float32),
        mesh=mesh,
        scratch_types=[
            pltpu.VMEM((K, B_EDGE), jnp.int32),
            pltpu.VMEM((NP,), jnp.float32),
            pltpu.VMEM((R,), jnp.float32),
            pltpu.VMEM((R,), jnp.float32),
            pltpu.VMEM_SHARED((NS, NP), jnp.float32),
            pltpu.SemaphoreType.DMA,
        ],
    )
    def deg_kernel(dstl_hbm, out_hbm, idx_d, hist, accb, tmpb, shist, sem):
        c = lax.axis_index("c")
        s = lax.axis_index("s")
        wid = c * NS + s
        pltpu.async_copy(dstl_hbm.at[wid], idx_d, sem).wait()

        zero16 = jnp.zeros((LANES,), jnp.float32)

        @pl.loop(0, NP // LANES)
        def _(i):
            hist[pl.ds(i * LANES, LANES)] = zero16

        ones = jnp.ones((LANES,), jnp.float32)

        @pl.loop(0, K)
        def _(k):
            for j in range(B_EDGE // LANES):
                idxv = idx_d[k, pl.ds(j * LANES, LANES)]
                plsc.addupdate_scatter(hist, [idxv], ones)

        pltpu.sync_copy(hist, shist.at[s])
        plsc.subcore_barrier()

        pltpu.sync_copy(shist.at[0, pl.ds(s * R, R)], accb)
        for t in range(1, NS):
            pltpu.sync_copy(shist.at[t, pl.ds(s * R, R)], tmpb)

            @pl.loop(0, R // LANES)
            def _(j):
                sl = pl.ds(j * LANES, LANES)
                accb[sl] = accb[sl] + tmpb[sl]

        pltpu.sync_copy(accb, out_hbm.at[c, pl.ds(s * R, R)])

    return deg_kernel


def _make_agg_kernel(NP, K, D):
    R = NP // NS
    ZR = 64
    mesh = plsc.VectorSubcoreMesh(core_axis_name="c", subcore_axis_name="s")

    @functools.partial(
        pl.kernel,
        out_type=jax.ShapeDtypeStruct((NC, NP, D), jnp.float32),
        mesh=mesh,
        scratch_types=[
            pltpu.VMEM((K, B_EDGE), jnp.int32),
            pltpu.VMEM((K, B_EDGE), jnp.int32),
            pltpu.VMEM((NBUF, B_EDGE, D), jnp.float32),
            pltpu.VMEM((64, D), jnp.float32),
            pltpu.VMEM_SHARED((NP, D), jnp.float32),
            pltpu.SemaphoreType.DMA((NBUF,)),
            pltpu.SemaphoreType.DMA,
        ],
    )
    def agg_kernel(hs_hbm, srcl_hbm, dstl_hbm, out_hbm,
                   idx_s, idx_d, rows, zblk, acc, gsems, sem):
        c = lax.axis_index("c")
        s = lax.axis_index("s")
        wid = c * NS + s
        pltpu.async_copy(srcl_hbm.at[wid], idx_s, sem).wait()
        pltpu.async_copy(dstl_hbm.at[wid], idx_d, sem).wait()

        zero16 = jnp.zeros((LANES,), jnp.float32)
        ZR = 64

        @pl.loop(0, ZR)
        def _(r):
            for cc in range(D // LANES):
                zblk[r, pl.ds(cc * LANES, LANES)] = zero16

        @pl.loop(0, R // ZR)
        def _(t):
            pltpu.sync_copy(zblk, acc.at[pl.ds(s * R + t * ZR, ZR)])

        plsc.subcore_barrier()

        for b in range(NBUF):
            pltpu.async_copy(hs_hbm.at[idx_s.at[b]], rows.at[b], gsems.at[b])

        @pl.loop(0, K, step=NBUF)
        def _(k0):
            for b in range(NBUF):
                kk = k0 + b
                pltpu.make_async_copy(
                    hs_hbm.at[idx_s.at[kk]], rows.at[b], gsems.at[b]).wait()
                pltpu.sync_copy(rows.at[b], acc.at[idx_d.at[kk]], add=True)

                @pl.when(kk + NBUF < K)
                def _():
                    pltpu.async_copy(
                        hs_hbm.at[idx_s.at[kk + NBUF]], rows.at[b],
                        gsems.at[b])

        plsc.subcore_barrier()
        pltpu.sync_copy(acc.at[pl.ds(s * R, R)],
                        out_hbm.at[c].at[pl.ds(s * R, R)])

    return agg_kernel


def _tc_matmul(x, w, blk=512):
    NP, KD = x.shape
    D = w.shape[1]

    def body(x_ref, w_ref, o_ref):
        o_ref[...] = jnp.dot(x_ref[...], w_ref[...],
                             preferred_element_type=jnp.float32)

    return pl.pallas_call(
        body,
        grid=(NP // blk,),
        in_specs=[
            pl.BlockSpec((blk, KD), lambda i: (i, 0)),
            pl.BlockSpec((KD, D), lambda i: (0, 0)),
        ],
        out_specs=pl.BlockSpec((blk, D), lambda i: (i, 0)),
        out_shape=jax.ShapeDtypeStruct((NP, D), jnp.float32),
    )(x, w)


def _dinv_of(p0, p1):
    return lax.rsqrt(p0 + p1 + 1.0)


def _tc_scale(p0, p1, h, blk=2048):
    NP, D = h.shape

    def body(p0_ref, p1_ref, h_ref, o_ref):
        dinv = _dinv_of(p0_ref[...], p1_ref[...])
        o_ref[...] = h_ref[...] * dinv

    return pl.pallas_call(
        body,
        grid=(NP // blk,),
        in_specs=[
            pl.BlockSpec((blk, 1), lambda i: (i, 0)),
            pl.BlockSpec((blk, 1), lambda i: (i, 0)),
            pl.BlockSpec((blk, D), lambda i: (i, 0)),
        ],
        out_specs=pl.BlockSpec((blk, D), lambda i: (i, 0)),
        out_shape=jax.ShapeDtypeStruct((NP, D), jnp.float32),
    )(p0, p1, h)


def _tc_mid(a0, a1, p0, p1, h1, b1, w2p, blk=1024):
    NP, D = h1.shape
    D2 = w2p.shape[1]

    def body(a0_ref, a1_ref, p0_ref, p1_ref, h1_ref, b1_ref, w2_ref,
             h2_ref, hs2_ref):
        dinv = _dinv_of(p0_ref[...], p1_ref[...])
        out1 = (a0_ref[...] + a1_ref[...]) * dinv \
            + h1_ref[...] * (dinv * dinv) + b1_ref[...]
        x1 = jnp.where(out1 >= 0, out1, 0.01 * out1)
        h2 = jnp.dot(x1, w2_ref[...], preferred_element_type=jnp.float32)
        h2_ref[...] = h2
        hs2_ref[...] = h2 * dinv

    return pl.pallas_call(
        body,
        grid=(NP // blk,),
        in_specs=[
            pl.BlockSpec((blk, D), lambda i: (i, 0)),
            pl.BlockSpec((blk, D), lambda i: (i, 0)),
            pl.BlockSpec((blk, 1), lambda i: (i, 0)),
            pl.BlockSpec((blk, 1), lambda i: (i, 0)),
            pl.BlockSpec((blk, D), lambda i: (i, 0)),
            pl.BlockSpec((1, D), lambda i: (0, 0)),
            pl.BlockSpec((D, D2), lambda i: (0, 0)),
        ],
        out_specs=[
            pl.BlockSpec((blk, D2), lambda i: (i, 0)),
            pl.BlockSpec((blk, D2), lambda i: (i, 0)),
        ],
        out_shape=[
            jax.ShapeDtypeStruct((NP, D2), jnp.float32),
            jax.ShapeDtypeStruct((NP, D2), jnp.float32),
        ],
    )(a0, a1, p0, p1, h1, b1, w2p)


def _tc_final(a0, a1, p0, p1, h2, b2p, ncls, blk=2048):
    NP, D2 = h2.shape

    def body(a0_ref, a1_ref, p0_ref, p1_ref, h2_ref, b2_ref, o_ref):
        dinv = _dinv_of(p0_ref[...], p1_ref[...])
        o = (a0_ref[...] + a1_ref[...]) * dinv \
            + h2_ref[...] * (dinv * dinv) + b2_ref[...]
        col = lax.broadcasted_iota(jnp.int32, (blk, D2), 1)
        valid = col < ncls
        om = jnp.where(valid, o, -1e30)
        m = jnp.max(om, axis=1, keepdims=True)
        e = jnp.where(valid, jnp.exp(om - m), 0.0)
        ssum = jnp.sum(e, axis=1, keepdims=True)
        o_ref[...] = e / ssum

    return pl.pallas_call(
        body,
        grid=(NP // blk,),
        in_specs=[
            pl.BlockSpec((blk, D2), lambda i: (i, 0)),
            pl.BlockSpec((blk, D2), lambda i: (i, 0)),
            pl.BlockSpec((blk, 1), lambda i: (i, 0)),
            pl.BlockSpec((blk, 1), lambda i: (i, 0)),
            pl.BlockSpec((blk, D2), lambda i: (i, 0)),
            pl.BlockSpec((1, D2), lambda i: (0, 0)),
        ],
        out_specs=pl.BlockSpec((blk, D2), lambda i: (i, 0)),
        out_shape=jax.ShapeDtypeStruct((NP, D2), jnp.float32),
    )(a0, a1, p0, p1, h2, b2p)


def kernel(x_embeddings, edge_index, W1, b1, W2, b2):
    N, F0 = x_embeddings.shape
    E = edge_index.shape[1]
    F1 = W1.shape[1]
    ncls = W2.shape[1]
    D2 = 16

    NP = _round_up(N + 1, NS * 64)
    K = _round_up(E, NW * B_EDGE * NBUF) // (NW * B_EDGE)
    EP = NW * K * B_EDGE

    xp = jnp.pad(x_embeddings, ((0, NP - N), (0, 0)))
    e32 = edge_index.astype(jnp.int32)
    pad_idx = jnp.full((EP - E,), N, jnp.int32)
    srcl = jnp.concatenate([e32[0], pad_idx]).reshape(NW, K, B_EDGE)
    dstl = jnp.concatenate([e32[1], pad_idx]).reshape(NW, K, B_EDGE)
    w2p = jnp.pad(W2, ((0, 0), (0, D2 - ncls)))
    b1r = b1.reshape(1, F1)
    b2r = jnp.pad(b2, (0, D2 - ncls)).reshape(1, D2)

    deg_parts = _make_deg_kernel(NP, K)(dstl)
    p0 = deg_parts[0].reshape(NP, 1)
    p1 = deg_parts[1].reshape(NP, 1)

    h1 = _tc_matmul(xp, W1)
    hs1 = _tc_scale(p0, p1, h1)

    acc1 = _make_agg_kernel(NP, K, F1)(hs1, srcl, dstl)

    h2, hs2 = _tc_mid(acc1[0], acc1[1], p0, p1, h1, b1r, w2p)

    acc2 = _make_agg_kernel(NP, K, D2)(hs2, srcl, dstl)

    out = _tc_final(acc2[0], acc2[1], p0, p1, h2, b2r, ncls)
    return out[:N, :ncls]


# speedup vs baseline: 23.3087x; 23.3087x over previous
import functools

import jax
import jax.numpy as jnp
from jax import lax
from jax.experimental import pallas as pl
from jax.experimental.pallas import tpu as pltpu
from jax.experimental.pallas import tpu_sc as plsc

NC = 2
NS = 16
NW = NC * NS
LANES = 16
B_EDGE = 128
NBUF = 2


def _round_up(x, m):
    return (x + m - 1) // m * m


_SC_PARAMS = pltpu.CompilerParams(needs_layout_passes=False,
                                  use_tc_tiling_on_sc=False)


def _make_deg_kernel(NP, K):
    R = NP // NS
    mesh = plsc.VectorSubcoreMesh(core_axis_name="c", subcore_axis_name="s")

    @functools.partial(
        pl.kernel,
        out_type=jax.ShapeDtypeStruct((NC, NP), jnp.float32),
        mesh=mesh,
        scratch_types=[
            pltpu.VMEM((K, B_EDGE), jnp.int32),
            pltpu.VMEM((NP,), jnp.float32),
            pltpu.VMEM((R,), jnp.float32),
            pltpu.VMEM((R,), jnp.float32),
            pltpu.VMEM_SHARED((NS, NP), jnp.float32),
            pltpu.SemaphoreType.DMA,
        ],
        compiler_params=_SC_PARAMS,
    )
    def deg_kernel(dstl_hbm, out_hbm, idx_d, hist, accb, tmpb, shist, sem):
        c = lax.axis_index("c")
        s = lax.axis_index("s")
        wid = c * NS + s
        pltpu.async_copy(dstl_hbm.at[wid], idx_d, sem).wait()

        zero16 = jnp.zeros((LANES,), jnp.float32)

        @pl.loop(0, NP // LANES)
        def _(i):
            hist[pl.ds(i * LANES, LANES)] = zero16

        ones = jnp.ones((LANES,), jnp.float32)

        @pl.loop(0, K)
        def _(k):
            for j in range(B_EDGE // LANES):
                idxv = idx_d[k, pl.ds(j * LANES, LANES)]
                plsc.addupdate_scatter(hist, [idxv], ones)

        pltpu.sync_copy(hist, shist.at[s])
        plsc.subcore_barrier()

        pltpu.sync_copy(shist.at[0, pl.ds(s * R, R)], accb)
        for t in range(1, NS):
            pltpu.sync_copy(shist.at[t, pl.ds(s * R, R)], tmpb)

            @pl.loop(0, R // LANES)
            def _(j):
                sl = pl.ds(j * LANES, LANES)
                accb[sl] = accb[sl] + tmpb[sl]

        pltpu.sync_copy(accb, out_hbm.at[c, pl.ds(s * R, R)])

    return deg_kernel


def _make_agg_kernel(NP, K, D):
    R = NP // NS
    ZR = 64
    mesh = plsc.VectorSubcoreMesh(core_axis_name="c", subcore_axis_name="s")

    @functools.partial(
        pl.kernel,
        out_type=jax.ShapeDtypeStruct((NC, NP, D), jnp.float32),
        mesh=mesh,
        scratch_types=[
            pltpu.VMEM((K, B_EDGE), jnp.int32),
            pltpu.VMEM((K, B_EDGE), jnp.int32),
            pltpu.VMEM((NBUF, B_EDGE, D), jnp.float32),
            pltpu.VMEM((64, D), jnp.float32),
            pltpu.VMEM_SHARED((NP, D), jnp.float32),
            pltpu.SemaphoreType.DMA((NBUF,)),
            pltpu.SemaphoreType.DMA,
        ],
        compiler_params=_SC_PARAMS,
    )
    def agg_kernel(hs_hbm, srcl_hbm, dstl_hbm, out_hbm,
                   idx_s, idx_d, rows, zblk, acc, gsems, sem):
        c = lax.axis_index("c")
        s = lax.axis_index("s")
        wid = c * NS + s
        pltpu.async_copy(srcl_hbm.at[wid], idx_s, sem).wait()
        pltpu.async_copy(dstl_hbm.at[wid], idx_d, sem).wait()

        zero16 = jnp.zeros((LANES,), jnp.float32)
        ZR = 64

        @pl.loop(0, ZR)
        def _(r):
            for cc in range(D // LANES):
                zblk[r, pl.ds(cc * LANES, LANES)] = zero16

        @pl.loop(0, R // ZR)
        def _(t):
            pltpu.sync_copy(zblk, acc.at[pl.ds(s * R + t * ZR, ZR)])

        plsc.subcore_barrier()

        for b in range(NBUF):
            pltpu.async_copy(hs_hbm.at[idx_s.at[b]], rows.at[b], gsems.at[b])

        @pl.loop(0, K, step=NBUF)
        def _(k0):
            for b in range(NBUF):
                kk = k0 + b
                pltpu.make_async_copy(
                    hs_hbm.at[idx_s.at[kk]], rows.at[b], gsems.at[b]).wait()
                pltpu.sync_copy(rows.at[b], acc.at[idx_d.at[kk]], add=True)

                @pl.when(kk + NBUF < K)
                def _():
                    pltpu.async_copy(
                        hs_hbm.at[idx_s.at[kk + NBUF]], rows.at[b],
                        gsems.at[b])

        plsc.subcore_barrier()
        pltpu.sync_copy(acc.at[pl.ds(s * R, R)],
                        out_hbm.at[c].at[pl.ds(s * R, R)])

    return agg_kernel


def _tc_matmul(x, w, blk=512):
    NP, KD = x.shape
    D = w.shape[1]

    def body(x_ref, w_ref, o_ref):
        o_ref[...] = jnp.dot(x_ref[...], w_ref[...],
                             preferred_element_type=jnp.float32)

    return pl.pallas_call(
        body,
        grid=(NP // blk,),
        in_specs=[
            pl.BlockSpec((blk, KD), lambda i: (i, 0)),
            pl.BlockSpec((KD, D), lambda i: (0, 0)),
        ],
        out_specs=pl.BlockSpec((blk, D), lambda i: (i, 0)),
        out_shape=jax.ShapeDtypeStruct((NP, D), jnp.float32),
    )(x, w)


def _dinv_of(p0, p1):
    return lax.rsqrt(p0 + p1 + 1.0)


def _tc_scale(p0, p1, h, blk=2048):
    NP, D = h.shape

    def body(p0_ref, p1_ref, h_ref, o_ref):
        dinv = _dinv_of(p0_ref[...], p1_ref[...])
        o_ref[...] = h_ref[...] * dinv

    return pl.pallas_call(
        body,
        grid=(NP // blk,),
        in_specs=[
            pl.BlockSpec((blk, 1), lambda i: (i, 0)),
            pl.BlockSpec((blk, 1), lambda i: (i, 0)),
            pl.BlockSpec((blk, D), lambda i: (i, 0)),
        ],
        out_specs=pl.BlockSpec((blk, D), lambda i: (i, 0)),
        out_shape=jax.ShapeDtypeStruct((NP, D), jnp.float32),
    )(p0, p1, h)


def _tc_mid(a0, a1, p0, p1, h1, b1, w2p, blk=1024):
    NP, D = h1.shape
    D2 = w2p.shape[1]

    def body(a0_ref, a1_ref, p0_ref, p1_ref, h1_ref, b1_ref, w2_ref,
             h2_ref, hs2_ref):
        dinv = _dinv_of(p0_ref[...], p1_ref[...])
        out1 = (a0_ref[...] + a1_ref[...]) * dinv \
            + h1_ref[...] * (dinv * dinv) + b1_ref[...]
        x1 = jnp.where(out1 >= 0, out1, 0.01 * out1)
        h2 = jnp.dot(x1, w2_ref[...], preferred_element_type=jnp.float32)
        h2_ref[...] = h2
        hs2_ref[...] = h2 * dinv

    return pl.pallas_call(
        body,
        grid=(NP // blk,),
        in_specs=[
            pl.BlockSpec((blk, D), lambda i: (i, 0)),
            pl.BlockSpec((blk, D), lambda i: (i, 0)),
            pl.BlockSpec((blk, 1), lambda i: (i, 0)),
            pl.BlockSpec((blk, 1), lambda i: (i, 0)),
            pl.BlockSpec((blk, D), lambda i: (i, 0)),
            pl.BlockSpec((1, D), lambda i: (0, 0)),
            pl.BlockSpec((D, D2), lambda i: (0, 0)),
        ],
        out_specs=[
            pl.BlockSpec((blk, D2), lambda i: (i, 0)),
            pl.BlockSpec((blk, D2), lambda i: (i, 0)),
        ],
        out_shape=[
            jax.ShapeDtypeStruct((NP, D2), jnp.float32),
            jax.ShapeDtypeStruct((NP, D2), jnp.float32),
        ],
    )(a0, a1, p0, p1, h1, b1, w2p)


def _tc_final(a0, a1, p0, p1, h2, b2p, ncls, blk=2048):
    NP, D2 = h2.shape

    def body(a0_ref, a1_ref, p0_ref, p1_ref, h2_ref, b2_ref, o_ref):
        dinv = _dinv_of(p0_ref[...], p1_ref[...])
        o = (a0_ref[...] + a1_ref[...]) * dinv \
            + h2_ref[...] * (dinv * dinv) + b2_ref[...]
        col = lax.broadcasted_iota(jnp.int32, (blk, D2), 1)
        valid = col < ncls
        om = jnp.where(valid, o, -1e30)
        m = jnp.max(om, axis=1, keepdims=True)
        e = jnp.where(valid, jnp.exp(om - m), 0.0)
        ssum = jnp.sum(e, axis=1, keepdims=True)
        o_ref[...] = e / ssum

    return pl.pallas_call(
        body,
        grid=(NP // blk,),
        in_specs=[
            pl.BlockSpec((blk, D2), lambda i: (i, 0)),
            pl.BlockSpec((blk, D2), lambda i: (i, 0)),
            pl.BlockSpec((blk, 1), lambda i: (i, 0)),
            pl.BlockSpec((blk, 1), lambda i: (i, 0)),
            pl.BlockSpec((blk, D2), lambda i: (i, 0)),
            pl.BlockSpec((1, D2), lambda i: (0, 0)),
        ],
        out_specs=pl.BlockSpec((blk, D2), lambda i: (i, 0)),
        out_shape=jax.ShapeDtypeStruct((NP, D2), jnp.float32),
    )(a0, a1, p0, p1, h2, b2p)


def kernel(x_embeddings, edge_index, W1, b1, W2, b2):
    N, F0 = x_embeddings.shape
    E = edge_index.shape[1]
    F1 = W1.shape[1]
    ncls = W2.shape[1]
    D2 = 16

    NP = _round_up(N + 1, NS * 64)
    K = _round_up(E, NW * B_EDGE * NBUF) // (NW * B_EDGE)
    EP = NW * K * B_EDGE

    xp = jnp.pad(x_embeddings, ((0, NP - N), (0, 0)))
    e32 = edge_index.astype(jnp.int32)
    pad_idx = jnp.full((EP - E,), N, jnp.int32)
    srcl = jnp.concatenate([e32[0], pad_idx]).reshape(NW, K, B_EDGE)
    dstl = jnp.concatenate([e32[1], pad_idx]).reshape(NW, K, B_EDGE)
    w2p = jnp.pad(W2, ((0, 0), (0, D2 - ncls)))
    b1r = b1.reshape(1, F1)
    b2r = jnp.pad(b2, (0, D2 - ncls)).reshape(1, D2)

    deg_parts = _make_deg_kernel(NP, K)(dstl)
    p0 = deg_parts[0].reshape(NP, 1)
    p1 = deg_parts[1].reshape(NP, 1)

    h1 = _tc_matmul(xp, W1)
    hs1 = _tc_scale(p0, p1, h1)

    acc1 = _make_agg_kernel(NP, K, F1)(hs1, srcl, dstl)

    h2, hs2 = _tc_mid(acc1[0], acc1[1], p0, p1, h1, b1r, w2p)

    acc2 = _make_agg_kernel(NP, K, D2)(hs2, srcl, dstl)

    out = _tc_final(acc2[0], acc2[1], p0, p1, h2, b2r, ncls)
    return out[:N, :ncls]


# speedup vs baseline: 43.1449x; 1.8510x over previous
import functools

import jax
import jax.numpy as jnp
from jax import lax
from jax.experimental import pallas as pl
from jax.experimental.pallas import tpu as pltpu
from jax.experimental.pallas import tpu_sc as plsc

NC = 2
NS = 16
NW = NC * NS
LANES = 16
B_EDGE = 80
NBUF = 2


def _round_up(x, m):
    return (x + m - 1) // m * m


_SC_PARAMS = pltpu.CompilerParams(needs_layout_passes=False,
                                  use_tc_tiling_on_sc=False)


def _make_deg_kernel(NP, K):
    R = NP // NS
    mesh = plsc.VectorSubcoreMesh(core_axis_name="c", subcore_axis_name="s")

    @functools.partial(
        pl.kernel,
        out_type=jax.ShapeDtypeStruct((NC, NP), jnp.float32),
        mesh=mesh,
        scratch_types=[
            pltpu.VMEM((K, B_EDGE), jnp.int32),
            pltpu.VMEM((NP,), jnp.float32),
            pltpu.VMEM((R,), jnp.float32),
            pltpu.VMEM((R,), jnp.float32),
            pltpu.VMEM_SHARED((NS, NP), jnp.float32),
            pltpu.SemaphoreType.DMA,
        ],
        compiler_params=_SC_PARAMS,
    )
    def deg_kernel(dstl_hbm, out_hbm, idx_d, hist, accb, tmpb, shist, sem):
        c = lax.axis_index("c")
        s = lax.axis_index("s")
        wid = c * NS + s
        pltpu.async_copy(dstl_hbm.at[wid], idx_d, sem).wait()

        zero16 = jnp.zeros((LANES,), jnp.float32)

        @pl.loop(0, NP // LANES)
        def _(i):
            hist[pl.ds(i * LANES, LANES)] = zero16

        ones = jnp.ones((LANES,), jnp.float32)

        @pl.loop(0, K)
        def _(k):
            for j in range(B_EDGE // LANES):
                idxv = idx_d[k, pl.ds(j * LANES, LANES)]
                plsc.addupdate_scatter(hist, [idxv], ones)

        pltpu.sync_copy(hist, shist.at[s])
        plsc.subcore_barrier()

        pltpu.sync_copy(shist.at[0, pl.ds(s * R, R)], accb)
        for t in range(1, NS):
            pltpu.sync_copy(shist.at[t, pl.ds(s * R, R)], tmpb)

            @pl.loop(0, R // LANES)
            def _(j):
                sl = pl.ds(j * LANES, LANES)
                accb[sl] = accb[sl] + tmpb[sl]

        pltpu.sync_copy(accb, out_hbm.at[c, pl.ds(s * R, R)])

    return deg_kernel


def _make_agg_kernel(NP, K, D):
    R = NP // NS
    ZR = 64
    mesh = plsc.VectorSubcoreMesh(core_axis_name="c", subcore_axis_name="s")

    @functools.partial(
        pl.kernel,
        out_type=jax.ShapeDtypeStruct((NC, NP, D), jnp.float32),
        mesh=mesh,
        scratch_types=[
            pltpu.VMEM((K, B_EDGE), jnp.int32),
            pltpu.VMEM((K, B_EDGE), jnp.int32),
            pltpu.VMEM((NBUF, B_EDGE, D), jnp.float32),
            pltpu.VMEM((ZR, D), jnp.float32),
            pltpu.VMEM_SHARED((NP, D), jnp.float32),
            pltpu.VMEM_SHARED((NP, D), jnp.float32),
            pltpu.SemaphoreType.DMA((NBUF,)),
            pltpu.SemaphoreType.DMA,
        ],
        compiler_params=_SC_PARAMS,
    )
    def agg_kernel(hs_hbm, srcl_hbm, dstl_hbm, out_hbm,
                   idx_s, idx_d, rows, zblk, acc, hs_s, gsems, sem):
        c = lax.axis_index("c")
        s = lax.axis_index("s")
        wid = c * NS + s
        pltpu.async_copy(srcl_hbm.at[wid], idx_s, sem).wait()
        pltpu.async_copy(dstl_hbm.at[wid], idx_d, sem).wait()
        pltpu.async_copy(hs_hbm.at[pl.ds(s * R, R)],
                         hs_s.at[pl.ds(s * R, R)], sem).wait()

        zero16 = jnp.zeros((LANES,), jnp.float32)

        @pl.loop(0, ZR)
        def _(r):
            for cc in range(D // LANES):
                zblk[r, pl.ds(cc * LANES, LANES)] = zero16

        @pl.loop(0, R // ZR)
        def _(t):
            pltpu.sync_copy(zblk, acc.at[pl.ds(s * R + t * ZR, ZR)])

        plsc.subcore_barrier()

        for b in range(NBUF):
            pltpu.async_copy(hs_s.at[idx_s.at[b]], rows.at[b], gsems.at[b])

        K2 = K - (K % NBUF)

        @pl.loop(0, K2, step=NBUF)
        def _(k0):
            for b in range(NBUF):
                kk = k0 + b
                pltpu.make_async_copy(
                    hs_s.at[idx_s.at[kk]], rows.at[b], gsems.at[b]).wait()
                pltpu.sync_copy(rows.at[b], acc.at[idx_d.at[kk]], add=True)

                @pl.when(kk + NBUF < K)
                def _():
                    pltpu.async_copy(
                        hs_s.at[idx_s.at[kk + NBUF]], rows.at[b],
                        gsems.at[b])

        for kk in range(K2, K):
            b = kk % NBUF
            pltpu.make_async_copy(
                hs_s.at[idx_s.at[kk]], rows.at[b], gsems.at[b]).wait()
            pltpu.sync_copy(rows.at[b], acc.at[idx_d.at[kk]], add=True)

        plsc.subcore_barrier()
        pltpu.sync_copy(acc.at[pl.ds(s * R, R)],
                        out_hbm.at[c].at[pl.ds(s * R, R)])

    return agg_kernel


def _tc_matmul(x, w, NP, blk=512):
    KD = x.shape[1]
    D = w.shape[1]

    def body(x_ref, w_ref, o_ref):
        o_ref[...] = jnp.dot(x_ref[...], w_ref[...],
                             preferred_element_type=jnp.float32)

    return pl.pallas_call(
        body,
        grid=(NP // blk,),
        in_specs=[
            pl.BlockSpec((blk, KD), lambda i: (i, 0)),
            pl.BlockSpec((KD, D), lambda i: (0, 0)),
        ],
        out_specs=pl.BlockSpec((blk, D), lambda i: (i, 0)),
        out_shape=jax.ShapeDtypeStruct((NP, D), jnp.float32),
    )(x, w)


def _dinv_of(p0, p1):
    return lax.rsqrt(p0 + p1 + 1.0)


def _tc_scale(p0, p1, h, blk=2048):
    NP, D = h.shape

    def body(p0_ref, p1_ref, h_ref, o_ref):
        dinv = _dinv_of(p0_ref[...], p1_ref[...])
        o_ref[...] = h_ref[...] * dinv

    return pl.pallas_call(
        body,
        grid=(NP // blk,),
        in_specs=[
            pl.BlockSpec((blk, 1), lambda i: (i, 0)),
            pl.BlockSpec((blk, 1), lambda i: (i, 0)),
            pl.BlockSpec((blk, D), lambda i: (i, 0)),
        ],
        out_specs=pl.BlockSpec((blk, D), lambda i: (i, 0)),
        out_shape=jax.ShapeDtypeStruct((NP, D), jnp.float32),
    )(p0, p1, h)


def _tc_mid(acc, p0, p1, h1, b1, w2p, blk=1024):
    NP, D = h1.shape
    D2 = w2p.shape[1]

    def body(acc_ref, p0_ref, p1_ref, h1_ref, b1_ref, w2_ref,
             h2_ref, hs2_ref):
        dinv = _dinv_of(p0_ref[...], p1_ref[...])
        out1 = (acc_ref[0] + acc_ref[1]) * dinv \
            + h1_ref[...] * (dinv * dinv) + b1_ref[...]
        x1 = jnp.where(out1 >= 0, out1, 0.01 * out1)
        h2 = jnp.dot(x1, w2_ref[...], preferred_element_type=jnp.float32)
        h2_ref[...] = h2
        hs2_ref[...] = h2 * dinv

    return pl.pallas_call(
        body,
        grid=(NP // blk,),
        in_specs=[
            pl.BlockSpec((2, blk, D), lambda i: (0, i, 0)),
            pl.BlockSpec((blk, 1), lambda i: (i, 0)),
            pl.BlockSpec((blk, 1), lambda i: (i, 0)),
            pl.BlockSpec((blk, D), lambda i: (i, 0)),
            pl.BlockSpec((1, D), lambda i: (0, 0)),
            pl.BlockSpec((D, D2), lambda i: (0, 0)),
        ],
        out_specs=[
            pl.BlockSpec((blk, D2), lambda i: (i, 0)),
            pl.BlockSpec((blk, D2), lambda i: (i, 0)),
        ],
        out_shape=[
            jax.ShapeDtypeStruct((NP, D2), jnp.float32),
            jax.ShapeDtypeStruct((NP, D2), jnp.float32),
        ],
    )(acc, p0, p1, h1, b1, w2p)


def _tc_final(acc, p0, p1, h2, b2p, N, ncls, blk=2000):
    NP, D2 = h2.shape

    def body(acc_ref, p0_ref, p1_ref, h2_ref, b2_ref, o_ref):
        dinv = _dinv_of(p0_ref[...], p1_ref[...])
        o = (acc_ref[0] + acc_ref[1]) * dinv \
            + h2_ref[...] * (dinv * dinv) + b2_ref[...]
        col = lax.broadcasted_iota(jnp.int32, (blk, D2), 1)
        valid = col < ncls
        om = jnp.where(valid, o, -1e30)
        m = jnp.max(om, axis=1, keepdims=True)
        e = jnp.where(valid, jnp.exp(om - m), 0.0)
        ssum = jnp.sum(e, axis=1, keepdims=True)
        o_ref[...] = (e / ssum)[:, :ncls]

    return pl.pallas_call(
        body,
        grid=(N // blk,),
        in_specs=[
            pl.BlockSpec((2, blk, D2), lambda i: (0, i, 0)),
            pl.BlockSpec((blk, 1), lambda i: (i, 0)),
            pl.BlockSpec((blk, 1), lambda i: (i, 0)),
            pl.BlockSpec((blk, D2), lambda i: (i, 0)),
            pl.BlockSpec((1, D2), lambda i: (0, 0)),
        ],
        out_specs=pl.BlockSpec((blk, ncls), lambda i: (i, 0)),
        out_shape=jax.ShapeDtypeStruct((N, ncls), jnp.float32),
    )(acc, p0, p1, h2, b2p)


def kernel(x_embeddings, edge_index, W1, b1, W2, b2):
    N, F0 = x_embeddings.shape
    E = edge_index.shape[1]
    F1 = W1.shape[1]
    ncls = W2.shape[1]
    D2 = 16

    NP = _round_up(N + 1, NS * 64)
    assert E % (NW * B_EDGE) == 0
    K = E // (NW * B_EDGE)

    e32 = edge_index.astype(jnp.int32)
    srcl = e32[0].reshape(NW, K, B_EDGE)
    dstl = e32[1].reshape(NW, K, B_EDGE)
    w2p = jnp.pad(W2, ((0, 0), (0, D2 - ncls)))
    b1r = b1.reshape(1, F1)
    b2r = jnp.pad(b2, (0, D2 - ncls)).reshape(1, D2)

    deg_parts = _make_deg_kernel(NP, K)(dstl)
    p0 = deg_parts[0].reshape(NP, 1)
    p1 = deg_parts[1].reshape(NP, 1)

    h1 = _tc_matmul(x_embeddings, W1, NP)
    hs1 = _tc_scale(p0, p1, h1)

    acc1 = _make_agg_kernel(NP, K, F1)(hs1, srcl, dstl)

    h2, hs2 = _tc_mid(acc1, p0, p1, h1, b1r, w2p)

    acc2 = _make_agg_kernel(NP, K, D2)(hs2, srcl, dstl)

    return _tc_final(acc2, p0, p1, h2, b2r, N, ncls)


# speedup vs baseline: 45.9997x; 1.0662x over previous
import functools

import jax
import jax.numpy as jnp
from jax import lax
from jax.experimental import pallas as pl
from jax.experimental.pallas import tpu as pltpu
from jax.experimental.pallas import tpu_sc as plsc

NC = 2
NS = 16
NW = NC * NS
LANES = 16
B_EDGE = 100
B_DEG = 80
NBUF = 4
PRE = NBUF - 2


def _round_up(x, m):
    return (x + m - 1) // m * m


_SC_PARAMS = pltpu.CompilerParams(needs_layout_passes=False,
                                  use_tc_tiling_on_sc=False)


def _make_deg_kernel(NP, K):
    R = NP // NS
    mesh = plsc.VectorSubcoreMesh(core_axis_name="c", subcore_axis_name="s")

    @functools.partial(
        pl.kernel,
        out_type=jax.ShapeDtypeStruct((NC, NP), jnp.float32),
        mesh=mesh,
        scratch_types=[
            pltpu.VMEM((K, B_DEG), jnp.int32),
            pltpu.VMEM((NP,), jnp.float32),
            pltpu.VMEM((R,), jnp.float32),
            pltpu.VMEM((R,), jnp.float32),
            pltpu.VMEM_SHARED((NS, NP), jnp.float32),
            pltpu.SemaphoreType.DMA,
        ],
        compiler_params=_SC_PARAMS,
    )
    def deg_kernel(dstl_hbm, out_hbm, idx_d, hist, accb, tmpb, shist, sem):
        c = lax.axis_index("c")
        s = lax.axis_index("s")
        wid = c * NS + s
        pltpu.async_copy(dstl_hbm.at[wid], idx_d, sem).wait()

        zero16 = jnp.zeros((LANES,), jnp.float32)

        @pl.loop(0, NP // LANES)
        def _(i):
            hist[pl.ds(i * LANES, LANES)] = zero16

        ones = jnp.ones((LANES,), jnp.float32)

        @pl.loop(0, K)
        def _(k):
            for j in range(B_DEG // LANES):
                idxv = idx_d[k, pl.ds(j * LANES, LANES)]
                plsc.addupdate_scatter(hist, [idxv], ones)

        pltpu.sync_copy(hist, shist.at[s])
        plsc.subcore_barrier()

        pltpu.sync_copy(shist.at[0, pl.ds(s * R, R)], accb)
        for t in range(1, NS):
            pltpu.sync_copy(shist.at[t, pl.ds(s * R, R)], tmpb)

            @pl.loop(0, R // LANES)
            def _(j):
                sl = pl.ds(j * LANES, LANES)
                accb[sl] = accb[sl] + tmpb[sl]

        pltpu.sync_copy(accb, out_hbm.at[c, pl.ds(s * R, R)])

    return deg_kernel


def _make_agg_kernel(NP, K, D):
    R = NP // NS
    ZR = 16
    mesh = plsc.VectorSubcoreMesh(core_axis_name="c", subcore_axis_name="s")

    @functools.partial(
        pl.kernel,
        out_type=jax.ShapeDtypeStruct((NC, NP, D), jnp.float32),
        mesh=mesh,
        scratch_types=[
            pltpu.VMEM((K, B_EDGE), jnp.int32),
            pltpu.VMEM((K, B_EDGE), jnp.int32),
            pltpu.VMEM((NBUF, B_EDGE, D), jnp.float32),
            pltpu.VMEM((ZR, D), jnp.float32),
            pltpu.VMEM_SHARED((NP, D), jnp.float32),
            pltpu.VMEM_SHARED((NP, D), jnp.float32),
            pltpu.SemaphoreType.DMA((NBUF,)),
            pltpu.SemaphoreType.DMA((NBUF,)),
            pltpu.SemaphoreType.DMA,
        ],
        compiler_params=_SC_PARAMS,
    )
    def agg_kernel(hs_hbm, srcl_hbm, dstl_hbm, out_hbm,
                   idx_s, idx_d, rows, zblk, acc, hs_s, gsems, ssems, sem):
        c = lax.axis_index("c")
        s = lax.axis_index("s")
        wid = c * NS + s
        pltpu.async_copy(srcl_hbm.at[wid], idx_s, sem).wait()
        pltpu.async_copy(dstl_hbm.at[wid], idx_d, sem).wait()
        pltpu.async_copy(hs_hbm.at[pl.ds(s * R, R)],
                         hs_s.at[pl.ds(s * R, R)], sem).wait()

        zero16 = jnp.zeros((LANES,), jnp.float32)

        @pl.loop(0, ZR)
        def _(r):
            for cc in range(D // LANES):
                zblk[r, pl.ds(cc * LANES, LANES)] = zero16

        @pl.loop(0, R // ZR)
        def _(t):
            pltpu.sync_copy(zblk, acc.at[pl.ds(s * R + t * ZR, ZR)])

        plsc.subcore_barrier()

        assert K % NBUF == 0 and K >= 2 * NBUF

        def gath(j, slot):
            pltpu.async_copy(hs_s.at[idx_s.at[j]], rows.at[slot],
                             gsems.at[slot])

        def gath_wait(j, slot):
            pltpu.make_async_copy(hs_s.at[idx_s.at[j]], rows.at[slot],
                                  gsems.at[slot]).wait()

        def scat(j, slot):
            pltpu.async_copy(rows.at[slot], acc.at[idx_d.at[j]],
                             ssems.at[slot], add=True)

        def scat_wait(j, slot):
            pltpu.make_async_copy(rows.at[slot], acc.at[idx_d.at[j]],
                                  ssems.at[slot]).wait()

        for j in range(PRE):
            gath(j, j % NBUF)
        for j in range(2):
            gath(j + PRE, (j + PRE) % NBUF)
            gath_wait(j, j % NBUF)
            scat(j, j % NBUF)

        @pl.loop(2, K - PRE, step=NBUF)
        def _(k0):
            for b in range(NBUF):
                j = k0 + b
                scat_wait(j - 2, b)
                gath(j + PRE, b)
                gath_wait(j, (2 + b) % NBUF)
                scat(j, (2 + b) % NBUF)

        for j in range(K - PRE, K):
            gath_wait(j, j % NBUF)
            scat(j, j % NBUF)
        for j in range(K - NBUF, K):
            scat_wait(j, j % NBUF)

        plsc.subcore_barrier()
        pltpu.sync_copy(acc.at[pl.ds(s * R, R)],
                        out_hbm.at[c].at[pl.ds(s * R, R)])

    return agg_kernel


def _tc_matmul(x, w, NP, blk=512):
    KD = x.shape[1]
    D = w.shape[1]

    def body(x_ref, w_ref, o_ref):
        o_ref[...] = jnp.dot(x_ref[...], w_ref[...],
                             preferred_element_type=jnp.float32)

    return pl.pallas_call(
        body,
        grid=(NP // blk,),
        in_specs=[
            pl.BlockSpec((blk, KD), lambda i: (i, 0)),
            pl.BlockSpec((KD, D), lambda i: (0, 0)),
        ],
        out_specs=pl.BlockSpec((blk, D), lambda i: (i, 0)),
        out_shape=jax.ShapeDtypeStruct((NP, D), jnp.float32),
    )(x, w)


def _dinv_of(p0, p1):
    return lax.rsqrt(p0 + p1 + 1.0)


def _tc_scale(p0, p1, h, blk=2048):
    NP, D = h.shape

    def body(p0_ref, p1_ref, h_ref, o_ref):
        dinv = _dinv_of(p0_ref[...], p1_ref[...])
        o_ref[...] = h_ref[...] * dinv

    return pl.pallas_call(
        body,
        grid=(NP // blk,),
        in_specs=[
            pl.BlockSpec((blk, 1), lambda i: (i, 0)),
            pl.BlockSpec((blk, 1), lambda i: (i, 0)),
            pl.BlockSpec((blk, D), lambda i: (i, 0)),
        ],
        out_specs=pl.BlockSpec((blk, D), lambda i: (i, 0)),
        out_shape=jax.ShapeDtypeStruct((NP, D), jnp.float32),
    )(p0, p1, h)


def _tc_mid(acc, p0, p1, h1, b1, w2p, blk=1024):
    NP, D = h1.shape
    D2 = w2p.shape[1]

    def body(acc_ref, p0_ref, p1_ref, hs1_ref, b1_ref, w2_ref, hs2_ref):
        dinv = _dinv_of(p0_ref[...], p1_ref[...])
        out1 = (acc_ref[0] + acc_ref[1] + hs1_ref[...]) * dinv + b1_ref[...]
        x1 = jnp.where(out1 >= 0, out1, 0.01 * out1)
        h2 = jnp.dot(x1, w2_ref[...], preferred_element_type=jnp.float32)
        hs2_ref[...] = h2 * dinv

    return pl.pallas_call(
        body,
        grid=(NP // blk,),
        in_specs=[
            pl.BlockSpec((2, blk, D), lambda i: (0, i, 0)),
            pl.BlockSpec((blk, 1), lambda i: (i, 0)),
            pl.BlockSpec((blk, 1), lambda i: (i, 0)),
            pl.BlockSpec((blk, D), lambda i: (i, 0)),
            pl.BlockSpec((1, D), lambda i: (0, 0)),
            pl.BlockSpec((D, D2), lambda i: (0, 0)),
        ],
        out_specs=pl.BlockSpec((blk, D2), lambda i: (i, 0)),
        out_shape=jax.ShapeDtypeStruct((NP, D2), jnp.float32),
    )(acc, p0, p1, h1, b1, w2p)


def _tc_final(acc, p0, p1, h2, b2p, N, ncls, blk=2000):
    NP, D2 = h2.shape

    def body(acc_ref, p0_ref, p1_ref, hs2_ref, b2_ref, o_ref):
        dinv = _dinv_of(p0_ref[...], p1_ref[...])
        o = (acc_ref[0] + acc_ref[1] + hs2_ref[...]) * dinv + b2_ref[...]
        col = lax.broadcasted_iota(jnp.int32, (blk, D2), 1)
        valid = col < ncls
        om = jnp.where(valid, o, -1e30)
        m = jnp.max(om, axis=1, keepdims=True)
        e = jnp.where(valid, jnp.exp(om - m), 0.0)
        ssum = jnp.sum(e, axis=1, keepdims=True)
        o_ref[...] = (e / ssum)[:, :ncls]

    return pl.pallas_call(
        body,
        grid=(N // blk,),
        in_specs=[
            pl.BlockSpec((2, blk, D2), lambda i: (0, i, 0)),
            pl.BlockSpec((blk, 1), lambda i: (i, 0)),
            pl.BlockSpec((blk, 1), lambda i: (i, 0)),
            pl.BlockSpec((blk, D2), lambda i: (i, 0)),
            pl.BlockSpec((1, D2), lambda i: (0, 0)),
        ],
        out_specs=pl.BlockSpec((blk, ncls), lambda i: (i, 0)),
        out_shape=jax.ShapeDtypeStruct((N, ncls), jnp.float32),
    )(acc, p0, p1, h2, b2p)


def kernel(x_embeddings, edge_index, W1, b1, W2, b2):
    N, F0 = x_embeddings.shape
    E = edge_index.shape[1]
    F1 = W1.shape[1]
    ncls = W2.shape[1]
    D2 = 16

    NP = _round_up(N + 1, NS * 64)
    assert E % (NW * B_EDGE) == 0 and E % (NW * B_DEG) == 0
    K = E // (NW * B_EDGE)
    KD = E // (NW * B_DEG)

    e32 = edge_index.astype(jnp.int32)
    srcl = e32[0].reshape(NW, K, B_EDGE)
    dstl = e32[1].reshape(NW, K, B_EDGE)
    dstl_deg = e32[1].reshape(NW, KD, B_DEG)
    w2p = jnp.pad(W2, ((0, 0), (0, D2 - ncls)))
    b1r = b1.reshape(1, F1)
    b2r = jnp.pad(b2, (0, D2 - ncls)).reshape(1, D2)

    deg_parts = _make_deg_kernel(NP, KD)(dstl_deg)
    p0 = deg_parts[0].reshape(NP, 1)
    p1 = deg_parts[1].reshape(NP, 1)

    h1 = _tc_matmul(x_embeddings, W1, NP)
    hs1 = _tc_scale(p0, p1, h1)

    acc1 = _make_agg_kernel(NP, K, F1)(hs1, srcl, dstl)

    hs2 = _tc_mid(acc1, p0, p1, hs1, b1r, w2p)

    acc2 = _make_agg_kernel(NP, K, D2)(hs2, srcl, dstl)

    return _tc_final(acc2, p0, p1, hs2, b2r, N, ncls)


# speedup vs baseline: 48.6795x; 1.0583x over previous
import functools

import jax
import jax.numpy as jnp
from jax import lax
from jax.experimental import pallas as pl
from jax.experimental.pallas import tpu as pltpu
from jax.experimental.pallas import tpu_sc as plsc

NC = 2
NS = 16
NW = NC * NS
LANES = 16
B_EDGE = 100
B_DEG = 80
NBUF = 4
PRE = NBUF - 2


def _round_up(x, m):
    return (x + m - 1) // m * m


_SC_PARAMS = pltpu.CompilerParams(needs_layout_passes=False,
                                  use_tc_tiling_on_sc=False)


def _make_deg_kernel(NP, K):
    R = NP // NS
    mesh = plsc.VectorSubcoreMesh(core_axis_name="c", subcore_axis_name="s")

    @functools.partial(
        pl.kernel,
        out_type=jax.ShapeDtypeStruct((NC, NP), jnp.float32),
        mesh=mesh,
        scratch_types=[
            pltpu.VMEM((K, B_DEG), jnp.int32),
            pltpu.VMEM((NP,), jnp.float32),
            pltpu.VMEM((R,), jnp.float32),
            pltpu.VMEM((R,), jnp.float32),
            pltpu.VMEM_SHARED((NS, NP), jnp.float32),
            pltpu.SemaphoreType.DMA,
        ],
        compiler_params=_SC_PARAMS,
    )
    def deg_kernel(dstl_hbm, out_hbm, idx_d, hist, accb, tmpb, shist, sem):
        c = lax.axis_index("c")
        s = lax.axis_index("s")
        wid = c * NS + s
        pltpu.async_copy(dstl_hbm.at[wid], idx_d, sem).wait()

        zero16 = jnp.zeros((LANES,), jnp.float32)

        @pl.loop(0, NP // LANES)
        def _(i):
            hist[pl.ds(i * LANES, LANES)] = zero16

        ones = jnp.ones((LANES,), jnp.float32)

        @pl.loop(0, K)
        def _(k):
            for j in range(B_DEG // LANES):
                idxv = idx_d[k, pl.ds(j * LANES, LANES)]
                plsc.addupdate_scatter(hist, [idxv], ones)

        pltpu.sync_copy(hist, shist.at[s])
        plsc.subcore_barrier()

        pltpu.sync_copy(shist.at[0, pl.ds(s * R, R)], accb)
        for t in range(1, NS):
            pltpu.sync_copy(shist.at[t, pl.ds(s * R, R)], tmpb)

            @pl.loop(0, R // LANES)
            def _(j):
                sl = pl.ds(j * LANES, LANES)
                accb[sl] = accb[sl] + tmpb[sl]

        pltpu.sync_copy(accb, out_hbm.at[c, pl.ds(s * R, R)])

    return deg_kernel


def _make_agg_kernel(NP, K, D):
    R = NP // NS
    ZR = 16
    mesh = plsc.VectorSubcoreMesh(core_axis_name="c", subcore_axis_name="s")

    @functools.partial(
        pl.kernel,
        out_type=jax.ShapeDtypeStruct((NC, NP, D), jnp.float32),
        mesh=mesh,
        scratch_types=[
            pltpu.VMEM((K, B_EDGE), jnp.int32),
            pltpu.VMEM((K, B_EDGE), jnp.int32),
            pltpu.VMEM((NBUF, B_EDGE, D), jnp.float32),
            pltpu.VMEM((ZR, D), jnp.float32),
            pltpu.VMEM_SHARED((NP, D), jnp.float32),
            pltpu.VMEM_SHARED((NP, D), jnp.float32),
            pltpu.SemaphoreType.DMA((NBUF,)),
            pltpu.SemaphoreType.DMA((NBUF,)),
            pltpu.SemaphoreType.DMA,
        ],
        compiler_params=_SC_PARAMS,
    )
    def agg_kernel(hs_hbm, srcl_hbm, dstl_hbm, out_hbm,
                   idx_s, idx_d, rows, zblk, acc, hs_s, gsems, ssems, sem):
        c = lax.axis_index("c")
        s = lax.axis_index("s")
        wid = c * NS + s
        pltpu.async_copy(srcl_hbm.at[wid], idx_s, sem).wait()
        pltpu.async_copy(dstl_hbm.at[wid], idx_d, sem).wait()
        pltpu.async_copy(hs_hbm.at[pl.ds(s * R, R)],
                         hs_s.at[pl.ds(s * R, R)], sem).wait()

        zero16 = jnp.zeros((LANES,), jnp.float32)

        @pl.loop(0, ZR)
        def _(r):
            for cc in range(D // LANES):
                zblk[r, pl.ds(cc * LANES, LANES)] = zero16

        @pl.loop(0, R // ZR)
        def _(t):
            pltpu.sync_copy(zblk, acc.at[pl.ds(s * R + t * ZR, ZR)])

        plsc.subcore_barrier()

        assert K % NBUF == 0 and K >= 2 * NBUF

        def gath(j, slot):
            pltpu.async_copy(hs_s.at[idx_s.at[j]], rows.at[slot],
                             gsems.at[slot])

        def gath_wait(j, slot):
            pltpu.make_async_copy(hs_s.at[idx_s.at[j]], rows.at[slot],
                                  gsems.at[slot]).wait()

        def scat(j, slot):
            pltpu.async_copy(rows.at[slot], acc.at[idx_d.at[j]],
                             ssems.at[slot], add=True)

        def scat_wait(j, slot):
            pltpu.make_async_copy(rows.at[slot], acc.at[idx_d.at[j]],
                                  ssems.at[slot]).wait()

        for j in range(PRE):
            gath(j, j % NBUF)
        for j in range(2):
            gath(j + PRE, (j + PRE) % NBUF)
            gath_wait(j, j % NBUF)
            scat(j, j % NBUF)

        @pl.loop(2, K - PRE, step=NBUF)
        def _(k0):
            for b in range(NBUF):
                j = k0 + b
                scat_wait(j - 2, b)
                gath(j + PRE, b)
                gath_wait(j, (2 + b) % NBUF)
                scat(j, (2 + b) % NBUF)

        for j in range(K - PRE, K):
            gath_wait(j, j % NBUF)
            scat(j, j % NBUF)
        for j in range(K - NBUF, K):
            scat_wait(j, j % NBUF)

        plsc.subcore_barrier()
        pltpu.sync_copy(acc.at[pl.ds(s * R, R)],
                        out_hbm.at[c].at[pl.ds(s * R, R)])

    return agg_kernel


def _dinv_blk(deg_blk):
    return lax.rsqrt(deg_blk[0] + deg_blk[1] + 1.0)[:, None]


def _tc_mm_scale(deg, x, w, NP, blk=512):
    KD = x.shape[1]
    D = w.shape[1]

    def body(deg_ref, x_ref, w_ref, o_ref):
        h = jnp.dot(x_ref[...], w_ref[...],
                    preferred_element_type=jnp.float32)
        o_ref[...] = h * _dinv_blk(deg_ref[...])

    return pl.pallas_call(
        body,
        grid=(NP // blk,),
        in_specs=[
            pl.BlockSpec((2, blk), lambda i: (0, i)),
            pl.BlockSpec((blk, KD), lambda i: (i, 0)),
            pl.BlockSpec((KD, D), lambda i: (0, 0)),
        ],
        out_specs=pl.BlockSpec((blk, D), lambda i: (i, 0)),
        out_shape=jax.ShapeDtypeStruct((NP, D), jnp.float32),
    )(deg, x, w)


def _tc_mid(acc, deg, h1, b1, w2p, blk=1024):
    NP, D = h1.shape
    D2 = w2p.shape[1]

    def body(acc_ref, deg_ref, hs1_ref, b1_ref, w2_ref, hs2_ref):
        dinv = _dinv_blk(deg_ref[...])
        out1 = (acc_ref[0] + acc_ref[1] + hs1_ref[...]) * dinv + b1_ref[...]
        x1 = jnp.where(out1 >= 0, out1, 0.01 * out1)
        h2 = jnp.dot(x1, w2_ref[...], preferred_element_type=jnp.float32)
        hs2_ref[...] = h2 * dinv

    return pl.pallas_call(
        body,
        grid=(NP // blk,),
        in_specs=[
            pl.BlockSpec((2, blk, D), lambda i: (0, i, 0)),
            pl.BlockSpec((2, blk), lambda i: (0, i)),
            pl.BlockSpec((blk, D), lambda i: (i, 0)),
            pl.BlockSpec((1, D), lambda i: (0, 0)),
            pl.BlockSpec((D, D2), lambda i: (0, 0)),
        ],
        out_specs=pl.BlockSpec((blk, D2), lambda i: (i, 0)),
        out_shape=jax.ShapeDtypeStruct((NP, D2), jnp.float32),
    )(acc, deg, h1, b1, w2p)


def _tc_final(acc, deg, h2, b2p, ncls, blk=2048):
    NP, D2 = h2.shape

    def body(acc_ref, deg_ref, hs2_ref, b2_ref, o_ref):
        dinv = _dinv_blk(deg_ref[...])
        o = (acc_ref[0] + acc_ref[1] + hs2_ref[...]) * dinv + b2_ref[...]
        col = lax.broadcasted_iota(jnp.int32, (blk, D2), 1)
        valid = col < ncls
        om = jnp.where(valid, o, -1e30)
        m = jnp.max(om, axis=1, keepdims=True)
        e = jnp.where(valid, jnp.exp(om - m), 0.0)
        ssum = jnp.sum(e, axis=1, keepdims=True)
        o_ref[...] = (e / ssum)[:, :ncls]

    return pl.pallas_call(
        body,
        grid=(NP // blk,),
        in_specs=[
            pl.BlockSpec((2, blk, D2), lambda i: (0, i, 0)),
            pl.BlockSpec((2, blk), lambda i: (0, i)),
            pl.BlockSpec((blk, D2), lambda i: (i, 0)),
            pl.BlockSpec((1, D2), lambda i: (0, 0)),
        ],
        out_specs=pl.BlockSpec((blk, ncls), lambda i: (i, 0)),
        out_shape=jax.ShapeDtypeStruct((NP, ncls), jnp.float32),
    )(acc, deg, h2, b2p)


def kernel(x_embeddings, edge_index, W1, b1, W2, b2):
    N, F0 = x_embeddings.shape
    E = edge_index.shape[1]
    F1 = W1.shape[1]
    ncls = W2.shape[1]
    D2 = 16

    NP = _round_up(N + 1, NS * 64)
    assert E % (NW * B_EDGE) == 0 and E % (NW * B_DEG) == 0
    K = E // (NW * B_EDGE)
    KD = E // (NW * B_DEG)

    e32 = edge_index.astype(jnp.int32)
    srcl = e32[0].reshape(NW, K, B_EDGE)
    dstl = e32[1].reshape(NW, K, B_EDGE)
    dstl_deg = e32[1].reshape(NW, KD, B_DEG)
    w2p = jnp.pad(W2, ((0, 0), (0, D2 - ncls)))
    b1r = b1.reshape(1, F1)
    b2r = jnp.pad(b2, (0, D2 - ncls)).reshape(1, D2)

    deg_parts = _make_deg_kernel(NP, KD)(dstl_deg)

    hs1 = _tc_mm_scale(deg_parts, x_embeddings, W1, NP)

    acc1 = _make_agg_kernel(NP, K, F1)(hs1, srcl, dstl)

    hs2 = _tc_mid(acc1, deg_parts, hs1, b1r, w2p)

    acc2 = _make_agg_kernel(NP, K, D2)(hs2, srcl, dstl)

    return _tc_final(acc2, deg_parts, hs2, b2r, ncls)[:N]


# speedup vs baseline: 52.4868x; 1.0782x over previous
import functools

import jax
import jax.numpy as jnp
from jax import lax
from jax.experimental import pallas as pl
from jax.experimental.pallas import tpu as pltpu
from jax.experimental.pallas import tpu_sc as plsc

NC = 2
NS = 16
NW = NC * NS
LANES = 16
B_EDGE = 100
B_DEG = 80
NBUF = 4
PRE = NBUF - 2


def _round_up(x, m):
    return (x + m - 1) // m * m


_SC_PARAMS = pltpu.CompilerParams(needs_layout_passes=False,
                                  use_tc_tiling_on_sc=False)


def _make_deg_kernel(NP, K):
    R = NP // NS
    mesh = plsc.VectorSubcoreMesh(core_axis_name="c", subcore_axis_name="s")

    @functools.partial(
        pl.kernel,
        out_type=jax.ShapeDtypeStruct((NC, NP), jnp.float32),
        mesh=mesh,
        scratch_types=[
            pltpu.VMEM((K, B_DEG), jnp.int32),
            pltpu.VMEM((NP,), jnp.float32),
            pltpu.VMEM((R,), jnp.float32),
            pltpu.VMEM((R,), jnp.float32),
            pltpu.VMEM_SHARED((NS, NP), jnp.float32),
            pltpu.SemaphoreType.DMA,
        ],
        compiler_params=_SC_PARAMS,
    )
    def deg_kernel(dstl_hbm, out_hbm, idx_d, hist, accb, tmpb, shist, sem):
        c = lax.axis_index("c")
        s = lax.axis_index("s")
        wid = c * NS + s
        idx_cp = pltpu.async_copy(dstl_hbm.at[wid], idx_d, sem)

        zero16 = jnp.zeros((LANES,), jnp.float32)

        @pl.loop(0, NP // LANES)
        def _(i):
            hist[pl.ds(i * LANES, LANES)] = zero16

        ones = jnp.ones((LANES,), jnp.float32)
        idx_cp.wait()

        @pl.loop(0, K)
        def _(k):
            for j in range(B_DEG // LANES):
                idxv = idx_d[k, pl.ds(j * LANES, LANES)]
                plsc.addupdate_scatter(hist, [idxv], ones)

        pltpu.sync_copy(hist, shist.at[s])
        plsc.subcore_barrier()

        pltpu.sync_copy(shist.at[0, pl.ds(s * R, R)], accb)
        for t in range(1, NS):
            pltpu.sync_copy(shist.at[t, pl.ds(s * R, R)], tmpb)

            @pl.loop(0, R // LANES)
            def _(j):
                sl = pl.ds(j * LANES, LANES)
                accb[sl] = accb[sl] + tmpb[sl]

        pltpu.sync_copy(accb, out_hbm.at[c, pl.ds(s * R, R)])

    return deg_kernel


def _make_agg_kernel(NP, K, D):
    R = NP // NS
    ZR = 16
    mesh = plsc.VectorSubcoreMesh(core_axis_name="c", subcore_axis_name="s")

    @functools.partial(
        pl.kernel,
        out_type=jax.ShapeDtypeStruct((NC, NP, D), jnp.float32),
        mesh=mesh,
        scratch_types=[
            pltpu.VMEM((K, B_EDGE), jnp.int32),
            pltpu.VMEM((K, B_EDGE), jnp.int32),
            pltpu.VMEM((NBUF, B_EDGE, D), jnp.float32),
            pltpu.VMEM((ZR, D), jnp.float32),
            pltpu.VMEM_SHARED((NP, D), jnp.float32),
            pltpu.VMEM_SHARED((NP, D), jnp.float32),
            pltpu.SemaphoreType.DMA((NBUF,)),
            pltpu.SemaphoreType.DMA((NBUF,)),
            pltpu.SemaphoreType.DMA,
        ],
        compiler_params=_SC_PARAMS,
    )
    def agg_kernel(hs_hbm, srcl_hbm, dstl_hbm, out_hbm,
                   idx_s, idx_d, rows, zblk, acc, hs_s, gsems, ssems, sem):
        c = lax.axis_index("c")
        s = lax.axis_index("s")
        wid = c * NS + s
        cp_s = pltpu.async_copy(srcl_hbm.at[wid], idx_s, gsems.at[0])
        cp_d = pltpu.async_copy(dstl_hbm.at[wid], idx_d, gsems.at[1])
        cp_h = pltpu.async_copy(hs_hbm.at[pl.ds(s * R, R)],
                                hs_s.at[pl.ds(s * R, R)], sem)

        zero16 = jnp.zeros((LANES,), jnp.float32)

        @pl.loop(0, ZR)
        def _(r):
            for cc in range(D // LANES):
                zblk[r, pl.ds(cc * LANES, LANES)] = zero16

        @pl.loop(0, R // ZR)
        def _(t):
            pltpu.sync_copy(zblk, acc.at[pl.ds(s * R + t * ZR, ZR)])

        cp_s.wait()
        cp_d.wait()
        cp_h.wait()
        plsc.subcore_barrier()

        assert K % NBUF == 0 and K >= 2 * NBUF

        def gath(j, slot):
            pltpu.async_copy(hs_s.at[idx_s.at[j]], rows.at[slot],
                             gsems.at[slot])

        def gath_wait(j, slot):
            pltpu.make_async_copy(hs_s.at[idx_s.at[j]], rows.at[slot],
                                  gsems.at[slot]).wait()

        def scat(j, slot):
            pltpu.async_copy(rows.at[slot], acc.at[idx_d.at[j]],
                             ssems.at[slot], add=True)

        def scat_wait(j, slot):
            pltpu.make_async_copy(rows.at[slot], acc.at[idx_d.at[j]],
                                  ssems.at[slot]).wait()

        for j in range(PRE):
            gath(j, j % NBUF)
        for j in range(2):
            gath(j + PRE, (j + PRE) % NBUF)
            gath_wait(j, j % NBUF)
            scat(j, j % NBUF)

        @pl.loop(2, K - PRE, step=NBUF)
        def _(k0):
            for b in range(NBUF):
                j = k0 + b
                scat_wait(j - 2, b)
                gath(j + PRE, b)
                gath_wait(j, (2 + b) % NBUF)
                scat(j, (2 + b) % NBUF)

        for j in range(K - PRE, K):
            gath_wait(j, j % NBUF)
            scat(j, j % NBUF)
        for j in range(K - NBUF, K):
            scat_wait(j, j % NBUF)

        plsc.subcore_barrier()
        pltpu.sync_copy(acc.at[pl.ds(s * R, R)],
                        out_hbm.at[c].at[pl.ds(s * R, R)])

    return agg_kernel


def _dinv_blk(deg_blk):
    return lax.rsqrt(deg_blk[0] + deg_blk[1] + 1.0)[:, None]


def _tc_mm_scale(deg, x, w, NP, blk=1024):
    KD = x.shape[1]
    D = w.shape[1]

    def body(deg_ref, x_ref, w_ref, o_ref):
        h = jnp.dot(x_ref[...], w_ref[...],
                    preferred_element_type=jnp.float32)
        o_ref[...] = h * _dinv_blk(deg_ref[...])

    return pl.pallas_call(
        body,
        grid=(NP // blk,),
        in_specs=[
            pl.BlockSpec((2, blk), lambda i: (0, i)),
            pl.BlockSpec((blk, KD), lambda i: (i, 0)),
            pl.BlockSpec((KD, D), lambda i: (0, 0)),
        ],
        out_specs=pl.BlockSpec((blk, D), lambda i: (i, 0)),
        out_shape=jax.ShapeDtypeStruct((NP, D), jnp.float32),
    )(deg, x, w)


def _tc_mid(acc, deg, h1, b1, w2p, blk=1024):
    NP, D = h1.shape
    D2 = w2p.shape[1]

    def body(acc_ref, deg_ref, hs1_ref, b1_ref, w2_ref, hs2_ref):
        dinv = _dinv_blk(deg_ref[...])
        out1 = (acc_ref[0] + acc_ref[1] + hs1_ref[...]) * dinv + b1_ref[...]
        x1 = jnp.where(out1 >= 0, out1, 0.01 * out1)
        h2 = jnp.dot(x1, w2_ref[...], preferred_element_type=jnp.float32)
        hs2_ref[...] = h2 * dinv

    return pl.pallas_call(
        body,
        grid=(NP // blk,),
        in_specs=[
            pl.BlockSpec((2, blk, D), lambda i: (0, i, 0)),
            pl.BlockSpec((2, blk), lambda i: (0, i)),
            pl.BlockSpec((blk, D), lambda i: (i, 0)),
            pl.BlockSpec((1, D), lambda i: (0, 0)),
            pl.BlockSpec((D, D2), lambda i: (0, 0)),
        ],
        out_specs=pl.BlockSpec((blk, D2), lambda i: (i, 0)),
        out_shape=jax.ShapeDtypeStruct((NP, D2), jnp.float32),
    )(acc, deg, h1, b1, w2p)


def _tc_final(acc, deg, h2, b2p, ncls, blk=2048):
    NP, D2 = h2.shape

    def body(acc_ref, deg_ref, hs2_ref, b2_ref, o_ref):
        dinv = _dinv_blk(deg_ref[...])
        o = (acc_ref[0] + acc_ref[1] + hs2_ref[...]) * dinv + b2_ref[...]
        col = lax.broadcasted_iota(jnp.int32, (blk, D2), 1)
        valid = col < ncls
        om = jnp.where(valid, o, -1e30)
        m = jnp.max(om, axis=1, keepdims=True)
        e = jnp.where(valid, jnp.exp(om - m), 0.0)
        ssum = jnp.sum(e, axis=1, keepdims=True)
        o_ref[...] = (e / ssum)[:, :ncls]

    return pl.pallas_call(
        body,
        grid=(NP // blk,),
        in_specs=[
            pl.BlockSpec((2, blk, D2), lambda i: (0, i, 0)),
            pl.BlockSpec((2, blk), lambda i: (0, i)),
            pl.BlockSpec((blk, D2), lambda i: (i, 0)),
            pl.BlockSpec((1, D2), lambda i: (0, 0)),
        ],
        out_specs=pl.BlockSpec((blk, ncls), lambda i: (i, 0)),
        out_shape=jax.ShapeDtypeStruct((NP, ncls), jnp.float32),
    )(acc, deg, h2, b2p)


def kernel(x_embeddings, edge_index, W1, b1, W2, b2):
    N, F0 = x_embeddings.shape
    E = edge_index.shape[1]
    F1 = W1.shape[1]
    ncls = W2.shape[1]
    D2 = 16

    NP = _round_up(N + 1, NS * 64)
    assert E % (NW * B_EDGE) == 0 and E % (NW * B_DEG) == 0
    K = E // (NW * B_EDGE)
    KD = E // (NW * B_DEG)

    e32 = edge_index.astype(jnp.int32)
    srcl = e32[0].reshape(NW, K, B_EDGE)
    dstl = e32[1].reshape(NW, K, B_EDGE)
    dstl_deg = e32[1].reshape(NW, KD, B_DEG)
    w2p = jnp.pad(W2, ((0, 0), (0, D2 - ncls)))
    b1r = b1.reshape(1, F1)
    b2r = jnp.pad(b2, (0, D2 - ncls)).reshape(1, D2)

    deg_parts = _make_deg_kernel(NP, KD)(dstl_deg)

    hs1 = _tc_mm_scale(deg_parts, x_embeddings, W1, NP)

    acc1 = _make_agg_kernel(NP, K, F1)(hs1, srcl, dstl)

    hs2 = _tc_mid(acc1, deg_parts, hs1, b1r, w2p)

    acc2 = _make_agg_kernel(NP, K, D2)(hs2, srcl, dstl)

    return _tc_final(acc2, deg_parts, hs2, b2r, ncls)[:N]


# speedup vs baseline: 55.7443x; 1.0621x over previous
import functools

import jax
import jax.numpy as jnp
from jax import lax
from jax.experimental import pallas as pl
from jax.experimental.pallas import tpu as pltpu
from jax.experimental.pallas import tpu_sc as plsc

NC = 2
NS = 16
NW = NC * NS
LANES = 16
B_EDGE = 80
NBUF = 5
PRE = NBUF - 2


def _round_up(x, m):
    return (x + m - 1) // m * m


_SC_PARAMS = pltpu.CompilerParams(needs_layout_passes=False,
                                  use_tc_tiling_on_sc=False)


def _make_deg_kernel(NP, EW):
    R = NP // NS
    UNR = 5
    assert EW % (LANES * UNR) == 0
    mesh = plsc.VectorSubcoreMesh(core_axis_name="c", subcore_axis_name="s")

    @functools.partial(
        pl.kernel,
        out_type=jax.ShapeDtypeStruct((NC, NP), jnp.float32),
        mesh=mesh,
        scratch_types=[
            pltpu.VMEM((EW,), jnp.int32),
            pltpu.VMEM((NP,), jnp.float32),
            pltpu.VMEM((R,), jnp.float32),
            pltpu.VMEM((R,), jnp.float32),
            pltpu.VMEM_SHARED((NS, NP), jnp.float32),
            pltpu.SemaphoreType.DMA,
        ],
        compiler_params=_SC_PARAMS,
    )
    def deg_kernel(edges_hbm, out_hbm, idx_d, hist, accb, tmpb, shist, sem):
        c = lax.axis_index("c")
        s = lax.axis_index("s")
        wid = c * NS + s
        idx_cp = pltpu.async_copy(edges_hbm.at[1].at[pl.ds(wid * EW, EW)],
                                  idx_d, sem)

        zero16 = jnp.zeros((LANES,), jnp.float32)

        @pl.loop(0, NP // LANES)
        def _(i):
            hist[pl.ds(i * LANES, LANES)] = zero16

        ones = jnp.ones((LANES,), jnp.float32)
        idx_cp.wait()

        @pl.loop(0, EW // (LANES * UNR))
        def _(k):
            for j in range(UNR):
                idxv = idx_d[pl.ds((k * UNR + j) * LANES, LANES)]
                plsc.addupdate_scatter(hist, [idxv], ones)

        pltpu.sync_copy(hist, shist.at[s])
        plsc.subcore_barrier()

        pltpu.sync_copy(shist.at[0, pl.ds(s * R, R)], accb)
        for t in range(1, NS):
            pltpu.sync_copy(shist.at[t, pl.ds(s * R, R)], tmpb)

            @pl.loop(0, R // LANES)
            def _(j):
                sl = pl.ds(j * LANES, LANES)
                accb[sl] = accb[sl] + tmpb[sl]

        pltpu.sync_copy(accb, out_hbm.at[c, pl.ds(s * R, R)])

    return deg_kernel


def _make_agg_kernel(NP, EW, D):
    R = NP // NS
    ZR = 16
    K = EW // B_EDGE
    mesh = plsc.VectorSubcoreMesh(core_axis_name="c", subcore_axis_name="s")

    @functools.partial(
        pl.kernel,
        out_type=jax.ShapeDtypeStruct((NC, NP, D), jnp.float32),
        mesh=mesh,
        scratch_types=[
            pltpu.VMEM((EW,), jnp.int32),
            pltpu.VMEM((EW,), jnp.int32),
            pltpu.VMEM((NBUF, B_EDGE, D), jnp.float32),
            pltpu.VMEM((ZR, D), jnp.float32),
            pltpu.VMEM_SHARED((NP, D), jnp.float32),
            pltpu.VMEM_SHARED((NP, D), jnp.float32),
            pltpu.SemaphoreType.DMA((NBUF,)),
            pltpu.SemaphoreType.DMA((NBUF,)),
            pltpu.SemaphoreType.DMA,
        ],
        compiler_params=_SC_PARAMS,
    )
    def agg_kernel(hs_hbm, edges_hbm, out_hbm,
                   idx_s, idx_d, rows, zblk, acc, hs_s, gsems, ssems, sem):
        c = lax.axis_index("c")
        s = lax.axis_index("s")
        wid = c * NS + s
        cp_s = pltpu.async_copy(edges_hbm.at[0].at[pl.ds(wid * EW, EW)],
                                idx_s, gsems.at[0])
        cp_d = pltpu.async_copy(edges_hbm.at[1].at[pl.ds(wid * EW, EW)],
                                idx_d, gsems.at[1])
        cp_h = pltpu.async_copy(hs_hbm.at[pl.ds(s * R, R)],
                                hs_s.at[pl.ds(s * R, R)], sem)

        zero16 = jnp.zeros((LANES,), jnp.float32)

        @pl.loop(0, ZR)
        def _(r):
            for cc in range(D // LANES):
                zblk[r, pl.ds(cc * LANES, LANES)] = zero16

        @pl.loop(0, R // ZR)
        def _(t):
            pltpu.sync_copy(zblk, acc.at[pl.ds(s * R + t * ZR, ZR)])

        cp_s.wait()
        cp_d.wait()
        cp_h.wait()
        plsc.subcore_barrier()

        assert K % NBUF == 0 and K >= 2 * NBUF

        def bsl(j):
            return pl.ds(j * B_EDGE, B_EDGE)

        def gath(j, slot):
            pltpu.async_copy(hs_s.at[idx_s.at[bsl(j)]], rows.at[slot],
                             gsems.at[slot])

        def gath_wait(j, slot):
            pltpu.make_async_copy(hs_s.at[idx_s.at[bsl(j)]], rows.at[slot],
                                  gsems.at[slot]).wait()

        def scat(j, slot):
            pltpu.async_copy(rows.at[slot], acc.at[idx_d.at[bsl(j)]],
                             ssems.at[slot], add=True)

        def scat_wait(j, slot):
            pltpu.make_async_copy(rows.at[slot], acc.at[idx_d.at[bsl(j)]],
                                  ssems.at[slot]).wait()

        for j in range(PRE):
            gath(j, j % NBUF)
        for j in range(2):
            gath(j + PRE, (j + PRE) % NBUF)
            gath_wait(j, j % NBUF)
            scat(j, j % NBUF)

        @pl.loop(2, K - PRE, step=NBUF)
        def _(k0):
            for b in range(NBUF):
                j = k0 + b
                scat_wait(j - 2, b)
                gath(j + PRE, b)
                gath_wait(j, (2 + b) % NBUF)
                scat(j, (2 + b) % NBUF)

        for j in range(K - PRE, K):
            gath_wait(j, j % NBUF)
            scat(j, j % NBUF)
        for j in range(K - NBUF, K):
            scat_wait(j, j % NBUF)

        plsc.subcore_barrier()
        pltpu.sync_copy(acc.at[pl.ds(s * R, R)],
                        out_hbm.at[c].at[pl.ds(s * R, R)])

    return agg_kernel


def _dinv_blk(deg_blk, blk):
    del blk
    return lax.rsqrt(deg_blk[0] + deg_blk[1] + 1.0)[:, None]


def _tc_mm_scale(deg, x, w, NP, blk=1024):
    KD = x.shape[1]
    D = w.shape[1]

    def body(deg_ref, x_ref, w_ref, o_ref):
        h = jnp.dot(x_ref[...], w_ref[...],
                    preferred_element_type=jnp.float32)
        o_ref[...] = h * _dinv_blk(deg_ref[...], blk)

    return pl.pallas_call(
        body,
        grid=(NP // blk,),
        in_specs=[
            pl.BlockSpec((2, blk), lambda i: (0, i)),
            pl.BlockSpec((blk, KD), lambda i: (i, 0)),
            pl.BlockSpec((KD, D), lambda i: (0, 0)),
        ],
        out_specs=pl.BlockSpec((blk, D), lambda i: (i, 0)),
        out_shape=jax.ShapeDtypeStruct((NP, D), jnp.float32),
    )(deg, x, w)


def _tc_mid(acc, deg, h1, b1, w2p, blk=1024):
    NP, D = h1.shape
    D2 = w2p.shape[1]

    def body(acc_ref, deg_ref, hs1_ref, b1_ref, w2_ref, hs2_ref):
        dinv = _dinv_blk(deg_ref[...], blk)
        out1 = (acc_ref[0] + acc_ref[1] + hs1_ref[...]) * dinv + b1_ref[...]
        x1 = jnp.where(out1 >= 0, out1, 0.01 * out1)
        h2 = jnp.dot(x1, w2_ref[...], preferred_element_type=jnp.float32)
        hs2_ref[...] = h2 * dinv

    return pl.pallas_call(
        body,
        grid=(NP // blk,),
        in_specs=[
            pl.BlockSpec((2, blk, D), lambda i: (0, i, 0)),
            pl.BlockSpec((2, blk), lambda i: (0, i)),
            pl.BlockSpec((blk, D), lambda i: (i, 0)),
            pl.BlockSpec((1, D), lambda i: (0, 0)),
            pl.BlockSpec((D, D2), lambda i: (0, 0)),
        ],
        out_specs=pl.BlockSpec((blk, D2), lambda i: (i, 0)),
        out_shape=jax.ShapeDtypeStruct((NP, D2), jnp.float32),
    )(acc, deg, h1, b1, w2p)


def _tc_final(acc, deg, h2, b2p, ncls, blk=2048):
    NP, D2 = h2.shape

    def body(acc_ref, deg_ref, hs2_ref, b2_ref, o_ref):
        dinv = _dinv_blk(deg_ref[...], blk)
        o = (acc_ref[0] + acc_ref[1] + hs2_ref[...]) * dinv + b2_ref[...]
        col = lax.broadcasted_iota(jnp.int32, (blk, D2), 1)
        valid = col < ncls
        om = jnp.where(valid, o, -1e30)
        m = jnp.max(om, axis=1, keepdims=True)
        e = jnp.where(valid, jnp.exp(om - m), 0.0)
        ssum = jnp.sum(e, axis=1, keepdims=True)
        o_ref[...] = (e / ssum)[:, :ncls]

    return pl.pallas_call(
        body,
        grid=(NP // blk,),
        in_specs=[
            pl.BlockSpec((2, blk, D2), lambda i: (0, i, 0)),
            pl.BlockSpec((2, blk), lambda i: (0, i)),
            pl.BlockSpec((blk, D2), lambda i: (i, 0)),
            pl.BlockSpec((1, D2), lambda i: (0, 0)),
        ],
        out_specs=pl.BlockSpec((blk, ncls), lambda i: (i, 0)),
        out_shape=jax.ShapeDtypeStruct((NP, ncls), jnp.float32),
    )(acc, deg, h2, b2p)


def kernel(x_embeddings, edge_index, W1, b1, W2, b2):
    N, F0 = x_embeddings.shape
    E = edge_index.shape[1]
    F1 = W1.shape[1]
    ncls = W2.shape[1]
    D2 = 16

    NP = _round_up(N + 1, NS * 64)
    EW = E // NW
    assert E % NW == 0 and EW % B_EDGE == 0

    e32 = edge_index.astype(jnp.int32)
    w2p = jnp.pad(W2, ((0, 0), (0, D2 - ncls)))
    b1r = b1.reshape(1, F1)
    b2r = jnp.pad(b2, (0, D2 - ncls)).reshape(1, D2)

    deg3 = _make_deg_kernel(NP, EW)(e32)

    hs1 = _tc_mm_scale(deg3, x_embeddings, W1, NP)

    acc1 = _make_agg_kernel(NP, EW, F1)(hs1, e32)

    hs2 = _tc_mid(acc1, deg3, hs1, b1r, w2p)

    acc2 = _make_agg_kernel(NP, EW, D2)(hs2, e32)

    return _tc_final(acc2, deg3, hs2, b2r, ncls)[:N]


# speedup vs baseline: 56.2290x; 1.0087x over previous
import functools

import jax
import jax.numpy as jnp
from jax import lax
from jax.experimental import pallas as pl
from jax.experimental.pallas import tpu as pltpu
from jax.experimental.pallas import tpu_sc as plsc

NC = 2
NS = 16
NW = NC * NS
LANES = 16
B_EDGE = 80
NBUF = 5
PRE = NBUF - 2


def _round_up(x, m):
    return (x + m - 1) // m * m


_SC_PARAMS = pltpu.CompilerParams(needs_layout_passes=False,
                                  use_tc_tiling_on_sc=False)


def _make_deg_kernel(NP, EW):
    R = NP // NS
    UNR = 5
    assert EW % (LANES * UNR) == 0
    mesh = plsc.VectorSubcoreMesh(core_axis_name="c", subcore_axis_name="s")

    @functools.partial(
        pl.kernel,
        out_type=jax.ShapeDtypeStruct((NC, NP), jnp.float32),
        mesh=mesh,
        scratch_types=[
            pltpu.VMEM((EW,), jnp.int32),
            pltpu.VMEM((NP,), jnp.float32),
            pltpu.VMEM((R,), jnp.float32),
            pltpu.VMEM((R,), jnp.float32),
            pltpu.VMEM_SHARED((NS, NP), jnp.float32),
            pltpu.SemaphoreType.DMA,
        ],
        compiler_params=_SC_PARAMS,
    )
    def deg_kernel(edges_hbm, out_hbm, idx_d, hist, accb, tmpb, shist, sem):
        c = lax.axis_index("c")
        s = lax.axis_index("s")
        wid = c * NS + s
        idx_cp = pltpu.async_copy(edges_hbm.at[1].at[pl.ds(wid * EW, EW)],
                                  idx_d, sem)

        zero16 = jnp.zeros((LANES,), jnp.float32)

        @pl.loop(0, NP // LANES)
        def _(i):
            hist[pl.ds(i * LANES, LANES)] = zero16

        ones = jnp.ones((LANES,), jnp.float32)
        idx_cp.wait()

        @pl.loop(0, EW // (LANES * UNR))
        def _(k):
            for j in range(UNR):
                idxv = idx_d[pl.ds((k * UNR + j) * LANES, LANES)]
                plsc.addupdate_scatter(hist, [idxv], ones)

        pltpu.sync_copy(hist, shist.at[s])
        plsc.subcore_barrier()

        pltpu.sync_copy(shist.at[0, pl.ds(s * R, R)], accb)
        for t in range(1, NS):
            pltpu.sync_copy(shist.at[t, pl.ds(s * R, R)], tmpb)

            @pl.loop(0, R // LANES)
            def _(j):
                sl = pl.ds(j * LANES, LANES)
                accb[sl] = accb[sl] + tmpb[sl]

        pltpu.sync_copy(accb, out_hbm.at[c, pl.ds(s * R, R)])

    return deg_kernel


def _make_agg_kernel(NP, EW, D):
    R = NP // NS
    ZR = 16
    K = EW // B_EDGE
    mesh = plsc.VectorSubcoreMesh(core_axis_name="c", subcore_axis_name="s")

    @functools.partial(
        pl.kernel,
        out_type=jax.ShapeDtypeStruct((NC, NP, D), jnp.float32),
        mesh=mesh,
        scratch_types=[
            pltpu.VMEM((EW,), jnp.int32),
            pltpu.VMEM((EW,), jnp.int32),
            pltpu.VMEM((NBUF, B_EDGE, D), jnp.float32),
            pltpu.VMEM((ZR, D), jnp.float32),
            pltpu.VMEM_SHARED((NP, D), jnp.float32),
            pltpu.VMEM_SHARED((NP, D), jnp.float32),
            pltpu.SemaphoreType.DMA((NBUF,)),
            pltpu.SemaphoreType.DMA((NBUF,)),
            pltpu.SemaphoreType.DMA,
        ],
        compiler_params=_SC_PARAMS,
    )
    def agg_kernel(hs_hbm, edges_hbm, out_hbm,
                   idx_s, idx_d, rows, zblk, acc, hs_s, gsems, ssems, sem):
        c = lax.axis_index("c")
        s = lax.axis_index("s")
        wid = c * NS + s
        cp_s = pltpu.async_copy(edges_hbm.at[0].at[pl.ds(wid * EW, EW)],
                                idx_s, gsems.at[0])
        cp_d = pltpu.async_copy(edges_hbm.at[1].at[pl.ds(wid * EW, EW)],
                                idx_d, gsems.at[1])
        cp_h = pltpu.async_copy(hs_hbm.at[pl.ds(s * R, R)],
                                hs_s.at[pl.ds(s * R, R)], sem)

        zero16 = jnp.zeros((LANES,), jnp.float32)

        @pl.loop(0, ZR)
        def _(r):
            for cc in range(D // LANES):
                zblk[r, pl.ds(cc * LANES, LANES)] = zero16

        @pl.loop(0, R // ZR)
        def _(t):
            pltpu.sync_copy(zblk, acc.at[pl.ds(s * R + t * ZR, ZR)])

        cp_s.wait()
        cp_d.wait()
        cp_h.wait()
        plsc.subcore_barrier()

        assert K % NBUF == 0 and K >= 2 * NBUF

        def bsl(j):
            return pl.ds(j * B_EDGE, B_EDGE)

        def gath(j, slot):
            pltpu.async_copy(hs_s.at[idx_s.at[bsl(j)]], rows.at[slot],
                             gsems.at[slot])

        def gath_wait(j, slot):
            pltpu.make_async_copy(hs_s.at[idx_s.at[bsl(j)]], rows.at[slot],
                                  gsems.at[slot]).wait()

        def scat(j, slot):
            pltpu.async_copy(rows.at[slot], acc.at[idx_d.at[bsl(j)]],
                             ssems.at[slot], add=True)

        def scat_wait(j, slot):
            pltpu.make_async_copy(rows.at[slot], acc.at[idx_d.at[bsl(j)]],
                                  ssems.at[slot]).wait()

        for j in range(PRE):
            gath(j, j % NBUF)
        for j in range(2):
            gath(j + PRE, (j + PRE) % NBUF)
            gath_wait(j, j % NBUF)
            scat(j, j % NBUF)

        @pl.loop(2, K - PRE, step=NBUF)
        def _(k0):
            for b in range(NBUF):
                j = k0 + b
                scat_wait(j - 2, b)
                gath(j + PRE, b)
                gath_wait(j, (2 + b) % NBUF)
                scat(j, (2 + b) % NBUF)

        for j in range(K - PRE, K):
            gath_wait(j, j % NBUF)
            scat(j, j % NBUF)
        for j in range(K - NBUF, K):
            scat_wait(j, j % NBUF)

        plsc.subcore_barrier()
        pltpu.sync_copy(acc.at[pl.ds(s * R, R)],
                        out_hbm.at[c].at[pl.ds(s * R, R)])

    return agg_kernel


def _dinv_blk(deg_blk, blk):
    del blk
    return lax.rsqrt(deg_blk[0] + deg_blk[1] + 1.0)[:, None]


def _tc_mm_scale(deg, x, w, NP, blk=1024):
    KD = x.shape[1]
    D = w.shape[1]

    def body(deg_ref, x_ref, w_ref, o_ref):
        h = jnp.dot(x_ref[...], w_ref[...],
                    preferred_element_type=jnp.float32)
        o_ref[...] = h * _dinv_blk(deg_ref[...], blk)

    return pl.pallas_call(
        body,
        grid=(NP // blk,),
        in_specs=[
            pl.BlockSpec((2, blk), lambda i: (0, i)),
            pl.BlockSpec((blk, KD), lambda i: (i, 0)),
            pl.BlockSpec((KD, D), lambda i: (0, 0)),
        ],
        out_specs=pl.BlockSpec((blk, D), lambda i: (i, 0)),
        out_shape=jax.ShapeDtypeStruct((NP, D), jnp.float32),
    )(deg, x, w)


def _tc_mid(acc, deg, h1, b1, w2p, blk=1024):
    NP, D = h1.shape
    D2 = w2p.shape[1]

    def body(acc_ref, deg_ref, hs1_ref, b1_ref, w2_ref, hs2_ref):
        dinv = _dinv_blk(deg_ref[...], blk)
        out1 = (acc_ref[0] + acc_ref[1] + hs1_ref[...]) * dinv + b1_ref[...]
        x1 = jnp.where(out1 >= 0, out1, 0.01 * out1)
        h2 = jnp.dot(x1, w2_ref[...], preferred_element_type=jnp.float32)
        hs2_ref[...] = h2 * dinv

    return pl.pallas_call(
        body,
        grid=(NP // blk,),
        in_specs=[
            pl.BlockSpec((2, blk, D), lambda i: (0, i, 0)),
            pl.BlockSpec((2, blk), lambda i: (0, i)),
            pl.BlockSpec((blk, D), lambda i: (i, 0)),
            pl.BlockSpec((1, D), lambda i: (0, 0)),
            pl.BlockSpec((D, D2), lambda i: (0, 0)),
        ],
        out_specs=pl.BlockSpec((blk, D2), lambda i: (i, 0)),
        out_shape=jax.ShapeDtypeStruct((NP, D2), jnp.float32),
    )(acc, deg, h1, b1, w2p)


def _tc_final(acc, deg, h2, b2p, N, ncls, blk=2048):
    NP, D2 = h2.shape

    def body(acc_ref, deg_ref, hs2_ref, b2_ref, o_ref):
        dinv = _dinv_blk(deg_ref[...], blk)
        o = (acc_ref[0] + acc_ref[1] + hs2_ref[...]) * dinv + b2_ref[...]
        col = lax.broadcasted_iota(jnp.int32, (blk, D2), 1)
        valid = col < ncls
        om = jnp.where(valid, o, -1e30)
        m = jnp.max(om, axis=1, keepdims=True)
        e = jnp.where(valid, jnp.exp(om - m), 0.0)
        ssum = jnp.sum(e, axis=1, keepdims=True)
        o_ref[...] = (e / ssum)[:, :ncls]

    return pl.pallas_call(
        body,
        grid=(NP // blk,),
        in_specs=[
            pl.BlockSpec((2, blk, D2), lambda i: (0, i, 0)),
            pl.BlockSpec((2, blk), lambda i: (0, i)),
            pl.BlockSpec((blk, D2), lambda i: (i, 0)),
            pl.BlockSpec((1, D2), lambda i: (0, 0)),
        ],
        out_specs=pl.BlockSpec((blk, ncls), lambda i: (i, 0)),
        out_shape=jax.ShapeDtypeStruct((N, ncls), jnp.float32),
    )(acc, deg, h2, b2p)


def kernel(x_embeddings, edge_index, W1, b1, W2, b2):
    N, F0 = x_embeddings.shape
    E = edge_index.shape[1]
    F1 = W1.shape[1]
    ncls = W2.shape[1]
    D2 = 16

    NP = _round_up(N + 1, NS * 64)
    EW = E // NW
    assert E % NW == 0 and EW % B_EDGE == 0

    e32 = edge_index.astype(jnp.int32)
    w2p = jnp.pad(W2, ((0, 0), (0, D2 - ncls)))
    b1r = b1.reshape(1, F1)
    b2r = jnp.pad(b2, (0, D2 - ncls)).reshape(1, D2)

    deg3 = _make_deg_kernel(NP, EW)(e32)

    hs1 = _tc_mm_scale(deg3, x_embeddings, W1, NP)

    acc1 = _make_agg_kernel(NP, EW, F1)(hs1, e32)

    hs2 = _tc_mid(acc1, deg3, hs1, b1r, w2p)

    acc2 = _make_agg_kernel(NP, EW, D2)(hs2, e32)

    return _tc_final(acc2, deg3, hs2, b2r, N, ncls)


# speedup vs baseline: 56.7995x; 1.0101x over previous
import functools

import jax
import jax.numpy as jnp
from jax import lax
from jax.experimental import pallas as pl
from jax.experimental.pallas import tpu as pltpu
from jax.experimental.pallas import tpu_sc as plsc

NC = 2
NS = 16
NW = NC * NS
LANES = 16
B_EDGE = 80
NBUF = 5
PRE = NBUF - 2


def _round_up(x, m):
    return (x + m - 1) // m * m


_SC_PARAMS = pltpu.CompilerParams(needs_layout_passes=False,
                                  use_tc_tiling_on_sc=False)


def _make_deg_kernel(NP, EW):
    R = NP // NS
    UNR = 25
    assert EW % (LANES * UNR) == 0
    mesh = plsc.VectorSubcoreMesh(core_axis_name="c", subcore_axis_name="s")

    @functools.partial(
        pl.kernel,
        out_type=jax.ShapeDtypeStruct((NC, NP), jnp.float32),
        mesh=mesh,
        scratch_types=[
            pltpu.VMEM((EW,), jnp.int32),
            pltpu.VMEM((NP,), jnp.float32),
            pltpu.VMEM((R,), jnp.float32),
            pltpu.VMEM((R,), jnp.float32),
            pltpu.VMEM_SHARED((NS, NP), jnp.float32),
            pltpu.SemaphoreType.DMA,
        ],
        compiler_params=_SC_PARAMS,
    )
    def deg_kernel(edges_hbm, out_hbm, idx_d, hist, accb, tmpb, shist, sem):
        c = lax.axis_index("c")
        s = lax.axis_index("s")
        wid = c * NS + s
        idx_cp = pltpu.async_copy(edges_hbm.at[1].at[pl.ds(wid * EW, EW)],
                                  idx_d, sem)

        zero16 = jnp.zeros((LANES,), jnp.float32)

        @pl.loop(0, NP // LANES)
        def _(i):
            hist[pl.ds(i * LANES, LANES)] = zero16

        ones = jnp.ones((LANES,), jnp.float32)
        idx_cp.wait()

        @pl.loop(0, EW // (LANES * UNR))
        def _(k):
            for j in range(UNR):
                idxv = idx_d[pl.ds((k * UNR + j) * LANES, LANES)]
                plsc.addupdate_scatter(hist, [idxv], ones)

        pltpu.sync_copy(hist, shist.at[s])
        plsc.subcore_barrier()

        pltpu.sync_copy(shist.at[0, pl.ds(s * R, R)], accb)
        for t in range(1, NS):
            pltpu.sync_copy(shist.at[t, pl.ds(s * R, R)], tmpb)

            @pl.loop(0, R // LANES)
            def _(j):
                sl = pl.ds(j * LANES, LANES)
                accb[sl] = accb[sl] + tmpb[sl]

        pltpu.sync_copy(accb, out_hbm.at[c, pl.ds(s * R, R)])

    return deg_kernel


def _make_agg_kernel(NP, EW, D):
    R = NP // NS
    ZR = 16
    K = EW // B_EDGE
    mesh = plsc.VectorSubcoreMesh(core_axis_name="c", subcore_axis_name="s")

    @functools.partial(
        pl.kernel,
        out_type=jax.ShapeDtypeStruct((NC, NP, D), jnp.float32),
        mesh=mesh,
        scratch_types=[
            pltpu.VMEM((EW,), jnp.int32),
            pltpu.VMEM((EW,), jnp.int32),
            pltpu.VMEM((NBUF, B_EDGE, D), jnp.float32),
            pltpu.VMEM((ZR, D), jnp.float32),
            pltpu.VMEM_SHARED((NP, D), jnp.float32),
            pltpu.VMEM_SHARED((NP, D), jnp.float32),
            pltpu.SemaphoreType.DMA((NBUF,)),
            pltpu.SemaphoreType.DMA((NBUF,)),
            pltpu.SemaphoreType.DMA,
        ],
        compiler_params=_SC_PARAMS,
    )
    def agg_kernel(hs_hbm, edges_hbm, out_hbm,
                   idx_s, idx_d, rows, zblk, acc, hs_s, gsems, ssems, sem):
        c = lax.axis_index("c")
        s = lax.axis_index("s")
        wid = c * NS + s
        cp_s = pltpu.async_copy(edges_hbm.at[0].at[pl.ds(wid * EW, EW)],
                                idx_s, gsems.at[0])
        cp_d = pltpu.async_copy(edges_hbm.at[1].at[pl.ds(wid * EW, EW)],
                                idx_d, gsems.at[1])
        cp_h = pltpu.async_copy(hs_hbm.at[pl.ds(s * R, R)],
                                hs_s.at[pl.ds(s * R, R)], sem)

        zero16 = jnp.zeros((LANES,), jnp.float32)

        @pl.loop(0, ZR)
        def _(r):
            for cc in range(D // LANES):
                zblk[r, pl.ds(cc * LANES, LANES)] = zero16

        @pl.loop(0, R // ZR)
        def _(t):
            pltpu.sync_copy(zblk, acc.at[pl.ds(s * R + t * ZR, ZR)])

        cp_s.wait()
        cp_d.wait()
        cp_h.wait()
        plsc.subcore_barrier()

        assert K % NBUF == 0 and K >= 2 * NBUF

        def bsl(j):
            return pl.ds(j * B_EDGE, B_EDGE)

        def gath(j, slot):
            pltpu.async_copy(hs_s.at[idx_s.at[bsl(j)]], rows.at[slot],
                             gsems.at[slot])

        def gath_wait(j, slot):
            pltpu.make_async_copy(hs_s.at[idx_s.at[bsl(j)]], rows.at[slot],
                                  gsems.at[slot]).wait()

        def scat(j, slot):
            pltpu.async_copy(rows.at[slot], acc.at[idx_d.at[bsl(j)]],
                             ssems.at[slot], add=True)

        def scat_wait(j, slot):
            pltpu.make_async_copy(rows.at[slot], acc.at[idx_d.at[bsl(j)]],
                                  ssems.at[slot]).wait()

        for j in range(PRE):
            gath(j, j % NBUF)
        for j in range(2):
            gath(j + PRE, (j + PRE) % NBUF)
            gath_wait(j, j % NBUF)
            scat(j, j % NBUF)

        @pl.loop(2, K - PRE, step=NBUF)
        def _(k0):
            for b in range(NBUF):
                j = k0 + b
                scat_wait(j - 2, b)
                gath(j + PRE, b)
                gath_wait(j, (2 + b) % NBUF)
                scat(j, (2 + b) % NBUF)

        for j in range(K - PRE, K):
            gath_wait(j, j % NBUF)
            scat(j, j % NBUF)
        for j in range(K - NBUF, K):
            scat_wait(j, j % NBUF)

        plsc.subcore_barrier()
        pltpu.sync_copy(acc.at[pl.ds(s * R, R)],
                        out_hbm.at[c].at[pl.ds(s * R, R)])

    return agg_kernel


def _dinv_blk(deg_blk, blk):
    del blk
    return lax.rsqrt(deg_blk[0] + deg_blk[1] + 1.0)[:, None]


def _tc_mm_scale(deg, x, w, NP, blk=2048):
    KD = x.shape[1]
    D = w.shape[1]

    def body(deg_ref, x_ref, w_ref, o_ref):
        h = jnp.dot(x_ref[...], w_ref[...],
                    preferred_element_type=jnp.float32)
        o_ref[...] = h * _dinv_blk(deg_ref[...], blk)

    return pl.pallas_call(
        body,
        grid=(NP // blk,),
        in_specs=[
            pl.BlockSpec((2, blk), lambda i: (0, i)),
            pl.BlockSpec((blk, KD), lambda i: (i, 0)),
            pl.BlockSpec((KD, D), lambda i: (0, 0)),
        ],
        out_specs=pl.BlockSpec((blk, D), lambda i: (i, 0)),
        out_shape=jax.ShapeDtypeStruct((NP, D), jnp.float32),
    )(deg, x, w)


def _tc_mid(acc, deg, h1, b1, w2p, blk=1024):
    NP, D = h1.shape
    D2 = w2p.shape[1]

    def body(acc_ref, deg_ref, hs1_ref, b1_ref, w2_ref, hs2_ref):
        dinv = _dinv_blk(deg_ref[...], blk)
        out1 = (acc_ref[0] + acc_ref[1] + hs1_ref[...]) * dinv + b1_ref[...]
        x1 = jnp.where(out1 >= 0, out1, 0.01 * out1)
        h2 = jnp.dot(x1, w2_ref[...], preferred_element_type=jnp.float32)
        hs2_ref[...] = h2 * dinv

    return pl.pallas_call(
        body,
        grid=(NP // blk,),
        in_specs=[
            pl.BlockSpec((2, blk, D), lambda i: (0, i, 0)),
            pl.BlockSpec((2, blk), lambda i: (0, i)),
            pl.BlockSpec((blk, D), lambda i: (i, 0)),
            pl.BlockSpec((1, D), lambda i: (0, 0)),
            pl.BlockSpec((D, D2), lambda i: (0, 0)),
        ],
        out_specs=pl.BlockSpec((blk, D2), lambda i: (i, 0)),
        out_shape=jax.ShapeDtypeStruct((NP, D2), jnp.float32),
    )(acc, deg, h1, b1, w2p)


def _tc_final(acc, deg, h2, b2p, N, ncls, blk=2048):
    NP, D2 = h2.shape

    def body(acc_ref, deg_ref, hs2_ref, b2_ref, o_ref):
        dinv = _dinv_blk(deg_ref[...], blk)
        o = (acc_ref[0] + acc_ref[1] + hs2_ref[...]) * dinv + b2_ref[...]
        col = lax.broadcasted_iota(jnp.int32, (blk, D2), 1)
        valid = col < ncls
        om = jnp.where(valid, o, -1e30)
        m = jnp.max(om, axis=1, keepdims=True)
        e = jnp.where(valid, jnp.exp(om - m), 0.0)
        ssum = jnp.sum(e, axis=1, keepdims=True)
        o_ref[...] = (e / ssum)[:, :ncls]

    return pl.pallas_call(
        body,
        grid=(NP // blk,),
        in_specs=[
            pl.BlockSpec((2, blk, D2), lambda i: (0, i, 0)),
            pl.BlockSpec((2, blk), lambda i: (0, i)),
            pl.BlockSpec((blk, D2), lambda i: (i, 0)),
            pl.BlockSpec((1, D2), lambda i: (0, 0)),
        ],
        out_specs=pl.BlockSpec((blk, ncls), lambda i: (i, 0)),
        out_shape=jax.ShapeDtypeStruct((N, ncls), jnp.float32),
    )(acc, deg, h2, b2p)


def kernel(x_embeddings, edge_index, W1, b1, W2, b2):
    N, F0 = x_embeddings.shape
    E = edge_index.shape[1]
    F1 = W1.shape[1]
    ncls = W2.shape[1]
    D2 = 16

    NP = _round_up(N + 1, NS * 64)
    EW = E // NW
    assert E % NW == 0 and EW % B_EDGE == 0

    e32 = edge_index.astype(jnp.int32)
    w2p = jnp.pad(W2, ((0, 0), (0, D2 - ncls)))
    b1r = b1.reshape(1, F1)
    b2r = jnp.pad(b2, (0, D2 - ncls)).reshape(1, D2)

    deg3 = _make_deg_kernel(NP, EW)(e32)

    hs1 = _tc_mm_scale(deg3, x_embeddings, W1, NP)

    acc1 = _make_agg_kernel(NP, EW, F1)(hs1, e32)

    hs2 = _tc_mid(acc1, deg3, hs1, b1r, w2p)

    acc2 = _make_agg_kernel(NP, EW, D2)(hs2, e32)

    return _tc_final(acc2, deg3, hs2, b2r, N, ncls)
